# Initial kernel scaffold; baseline (speedup 1.0000x reference)
import jax
import jax.numpy as jnp
from jax.experimental import pallas as pl


def kernel(v, l, e, e_mask, vbi, vfc, vfb, W_feat, b_feat, W_lab, b_lab, W_msg, b_msg, W_upd, b_upd):
    raise NotImplementedError("write your pallas kernel here")


# speedup vs baseline: 4.3227x; 4.3227x over previous
import functools

import jax
import jax.numpy as jnp
import numpy as np
from jax import lax
from jax.experimental import pallas as pl
from jax.experimental.pallas import tpu as pltpu
from jax.experimental.pallas import tpu_sc as plsc

N = 50000
E = 800000
H = 32
D = 64
B = 16

BN = 400
NB = N // BN
WE = 128
NT = 16
CH = 3128
CH_LAST = N - (NT - 1) * CH


def _pe_table(d_model=32, max_len=20):
    pe = np.zeros((max_len, d_model), dtype=np.float32)
    position = np.arange(0, max_len, dtype=np.float32)[:, None]
    div_term = np.exp(np.arange(0, d_model, 2, dtype=np.float32) * (-np.log(10000.0) / d_model))
    pe[:, 0::2] = np.sin(position * div_term)
    pe[:, 1::2] = np.cos(position * div_term)
    return pe


_PE = _pe_table()


def _pool_body(vfc_ref, vbi_ref, out_ref):
    @pl.when(pl.program_id(0) == 0)
    def _():
        out_ref[...] = jnp.full((8, B), 127, jnp.int32)

    vfc_b = vfc_ref[0, 0, :]
    vbi_b = vbi_ref[0, 0, :]
    oh = vbi_b[:, None] == lax.broadcasted_iota(jnp.int32, (BN, B), 1)
    masked = jnp.where(oh, vfc_b[:, None], 127)
    colmin = jnp.min(masked, axis=0)
    out_ref[...] = jnp.minimum(out_ref[...], colmin[None, :])


def _enc_body(v_ref, l_ref, vfc_ref, vbi_ref, pool_ref, pe_ref,
              wf_ref, bf_ref, wl_ref, bl_ref, w2_ref,
              x_ref, pos16_ref, ylo_ref, yhi_ref):
    v_blk = v_ref[...]
    nonpos = jnp.concatenate([v_blk[:, 0:3], v_blk[:, 6:9]], axis=1)
    h = jnp.dot(nonpos, wf_ref[...], preferred_element_type=jnp.float32) + bf_ref[...]
    vfc_b = vfc_ref[0, 0, :]
    vbi_b = vbi_ref[0, 0, :]
    oh16 = vbi_b[:, None] == lax.broadcasted_iota(jnp.int32, (BN, B), 1)
    poolg = jnp.sum(jnp.where(oh16, pool_ref[0:1, :], 0), axis=1)
    lvl = vfc_b - poolg
    oh20 = (lvl[:, None] == lax.broadcasted_iota(jnp.int32, (BN, 20), 1)).astype(jnp.float32)
    pe_add = jnp.dot(oh20, pe_ref[...], preferred_element_type=jnp.float32)
    le = jnp.dot(l_ref[...], wl_ref[...], preferred_element_type=jnp.float32) + bl_ref[...]
    x = jnp.concatenate([h + pe_add, le], axis=1)
    x_ref[...] = x
    y = jnp.dot(x, w2_ref[...], preferred_element_type=jnp.float32)
    ylo_ref[...] = y[:, :32]
    yhi_ref[...] = y[:, 32:]
    pos = v_blk[:, 3:6]
    ones = jnp.ones((BN, 2), jnp.float32)
    pos16_ref[...] = jnp.concatenate([pos, ones, jnp.zeros((BN, 11), jnp.float32)], axis=1)


def _prep_body(r0_ref, r1_ref, v_ref, w3_ref, bm_ref, ed2_ref, ptn_ref):
    Rr = r0_ref[...] + r1_ref[...]
    Apos = Rr[:, 0:3]
    em = Rr[:, 3:4]
    degc = Rr[:, 4:5]
    invdeg = 1.0 / jnp.maximum(degc, 1.0)
    emn = em * invdeg
    pos = v_ref[...][:, 3:6]
    ptn = (jnp.dot(pos * em - Apos, w3_ref[...], preferred_element_type=jnp.float32)
           + em * bm_ref[...]) * invdeg
    ptn_ref[...] = ptn
    ed2_ref[...] = jnp.concatenate([emn, invdeg, jnp.zeros((BN, 14), jnp.float32)], axis=1)


def _post_body(x_ref, s0_ref, s1_ref, ed2_ref, ptn_ref,
               w1_ref, wu1_ref, wu2_ref, bu_ref, w2_ref,
               xn_ref, ylo_ref, yhi_ref):
    x = x_ref[...]
    S = jnp.concatenate([s0_ref[...], s1_ref[...]], axis=1)
    emn = ed2_ref[...][:, 0:1]
    invdeg = ed2_ref[...][:, 1:2]
    aggr = (jnp.dot(x, w1_ref[...], preferred_element_type=jnp.float32) * emn
            + S * invdeg + ptn_ref[...])
    upd = (jnp.dot(x, wu1_ref[...], preferred_element_type=jnp.float32)
           + jnp.dot(aggr, wu2_ref[...], preferred_element_type=jnp.float32)
           + bu_ref[...])
    xn = x + upd
    xn_ref[...] = xn
    y = jnp.dot(xn, w2_ref[...], preferred_element_type=jnp.float32)
    ylo_ref[...] = y[:, :32]
    yhi_ref[...] = y[:, 32:]


def _sc_mesh():
    return plsc.VectorSubcoreMesh(core_axis_name="c", subcore_axis_name="s")


def _copy_tile_rows(w, src_at, dst_at):
    @pl.when(w < NT - 1)
    def _():
        start = pl.multiple_of(w * CH, 8)
        pltpu.sync_copy(src_at(start, CH), dst_at(start, CH))

    @pl.when(w == NT - 1)
    def _():
        start = (NT - 1) * CH
        pltpu.sync_copy(src_at(start, CH_LAST), dst_at(start, CH_LAST))


def _scale_rows(rows_v, m_v, ncols, masked16=False):
    def grp(g, carry):
        m16 = m_v[pl.ds(g * 16, 16)]
        for e in range(16):
            eix = g * 16 + e
            sv = jnp.full((16,), 1.0, jnp.float32) * m16[e]
            if masked16:
                keep = lax.iota(jnp.int32, 16) < 4
                sv = jnp.where(keep, sv, 1.0)
                rows_v[eix, pl.ds(0, 16)] = rows_v[eix, pl.ds(0, 16)] * sv
            else:
                for f0 in range(0, ncols, 16):
                    rows_v[eix, pl.ds(f0, 16)] = rows_v[eix, pl.ds(f0, 16)] * sv
        return carry
    lax.fori_loop(0, WE // 16, grp, 0)


def _spmm_kernel(ylo_hbm, yhi_hbm, src_hbm, dst_hbm, m_hbm, z_hbm, out_hbm,
                 src_v, dst_v, m_v, rows_v, acc, sem):
    c = lax.axis_index("c")
    w = lax.axis_index("s")
    _copy_tile_rows(w, lambda s, n: z_hbm.at[pl.ds(s, n)],
                    lambda s, n: acc.at[pl.ds(s, n)])
    plsc.subcore_barrier()

    nwin = E // WE

    def win(jj, carry):
        widx = w + NT * jj

        @pl.when(widx < nwin)
        def _():
            base = widx * WE
            pltpu.sync_copy(src_hbm.at[pl.ds(base, WE)], src_v)
            pltpu.sync_copy(dst_hbm.at[pl.ds(base, WE)], dst_v)
            pltpu.sync_copy(m_hbm.at[pl.ds(base, WE)], m_v)

            @pl.when(c == 0)
            def _():
                pltpu.async_copy(ylo_hbm.at[src_v], rows_v, sem).wait()

            @pl.when(c == 1)
            def _():
                pltpu.async_copy(yhi_hbm.at[src_v], rows_v, sem).wait()

            _scale_rows(rows_v, m_v, 32)
            pltpu.sync_copy(rows_v, acc.at[dst_v], add=True)
        return carry

    lax.fori_loop(0, (nwin + NT - 1) // NT, win, 0)
    plsc.subcore_barrier()
    _copy_tile_rows(w, lambda s, n: acc.at[pl.ds(s, n)],
                    lambda s, n: out_hbm.at[pl.ds(c * N + s, n)])


def _pre_kernel(tab_hbm, src_hbm, dst_hbm, m_hbm, z_hbm, out_hbm,
                src_v, dst_v, m_v, rows_v, acc, sem):
    c = lax.axis_index("c")
    w = lax.axis_index("s")
    _copy_tile_rows(w, lambda s, n: z_hbm.at[pl.ds(s, n)],
                    lambda s, n: acc.at[pl.ds(s, n)])
    plsc.subcore_barrier()

    ehalf = E // 2
    nwin = ehalf // WE

    def win(jj, carry):
        widx = w + NT * jj

        @pl.when(widx < nwin)
        def _():
            base = c * ehalf + widx * WE
            pltpu.sync_copy(src_hbm.at[pl.ds(base, WE)], src_v)
            pltpu.sync_copy(dst_hbm.at[pl.ds(base, WE)], dst_v)
            pltpu.sync_copy(m_hbm.at[pl.ds(base, WE)], m_v)
            pltpu.async_copy(tab_hbm.at[src_v], rows_v, sem).wait()
            _scale_rows(rows_v, m_v, 16, masked16=True)
            pltpu.sync_copy(rows_v, acc.at[dst_v], add=True)
        return carry

    lax.fori_loop(0, (nwin + NT - 1) // NT, win, 0)
    plsc.subcore_barrier()
    _copy_tile_rows(w, lambda s, n: acc.at[pl.ds(s, n)],
                    lambda s, n: out_hbm.at[pl.ds(c * N + s, n)])


@functools.partial(jax.jit, static_argnums=())
def kernel(v, l, e, e_mask, vbi, vfc, vfb, W_feat, b_feat, W_lab, b_lab,
           W_msg, b_msg, W_upd, b_upd):
    src = e[0]
    dst = e[1]
    W1 = W_msg[0:D]
    W2 = W_msg[D:2 * D]
    W3 = W_msg[2 * D:]
    Wu1 = W_upd[0:D]
    Wu2 = W_upd[D:]
    pe = jnp.asarray(_PE)
    vfc3 = vfc.reshape(NB, 1, BN)
    vbi3 = vbi.reshape(NB, 1, BN)

    pool8 = pl.pallas_call(
        _pool_body,
        grid=(NB,),
        in_specs=[
            pl.BlockSpec((1, 1, BN), lambda i: (i, 0, 0)),
            pl.BlockSpec((1, 1, BN), lambda i: (i, 0, 0)),
        ],
        out_specs=pl.BlockSpec((8, B), lambda i: (0, 0)),
        out_shape=jax.ShapeDtypeStruct((8, B), jnp.int32),
    )(vfc3, vbi3)

    x, pos16, ylo, yhi = pl.pallas_call(
        _enc_body,
        grid=(NB,),
        in_specs=[
            pl.BlockSpec((BN, 9), lambda i: (i, 0)),
            pl.BlockSpec((BN, 8), lambda i: (i, 0)),
            pl.BlockSpec((1, 1, BN), lambda i: (i, 0, 0)),
            pl.BlockSpec((1, 1, BN), lambda i: (i, 0, 0)),
            pl.BlockSpec((8, B), lambda i: (0, 0)),
            pl.BlockSpec((20, 32), lambda i: (0, 0)),
            pl.BlockSpec((6, 32), lambda i: (0, 0)),
            pl.BlockSpec((1, 32), lambda i: (0, 0)),
            pl.BlockSpec((8, 32), lambda i: (0, 0)),
            pl.BlockSpec((1, 32), lambda i: (0, 0)),
            pl.BlockSpec((D, D), lambda i: (0, 0)),
        ],
        out_specs=[
            pl.BlockSpec((BN, D), lambda i: (i, 0)),
            pl.BlockSpec((BN, 16), lambda i: (i, 0)),
            pl.BlockSpec((BN, 32), lambda i: (i, 0)),
            pl.BlockSpec((BN, 32), lambda i: (i, 0)),
        ],
        out_shape=[
            jax.ShapeDtypeStruct((N, D), jnp.float32),
            jax.ShapeDtypeStruct((N, 16), jnp.float32),
            jax.ShapeDtypeStruct((N, 32), jnp.float32),
            jax.ShapeDtypeStruct((N, 32), jnp.float32),
        ],
    )(v, l, vfc3, vbi3, pool8, pe, W_feat, b_feat.reshape(1, 32),
      W_lab, b_lab.reshape(1, 32), W2)

    z16 = jnp.zeros((N, 16), jnp.float32)
    z32 = jnp.zeros((N, 32), jnp.float32)

    pre = functools.partial(
        pl.kernel,
        mesh=_sc_mesh(),
        compiler_params=pltpu.CompilerParams(use_tc_tiling_on_sc=False),
        out_type=jax.ShapeDtypeStruct((2 * N, 16), jnp.float32),
        scratch_types=[
            pltpu.VMEM((WE,), jnp.int32),
            pltpu.VMEM((WE,), jnp.int32),
            pltpu.VMEM((WE,), jnp.float32),
            pltpu.VMEM((WE, 16), jnp.float32),
            pltpu.VMEM_SHARED((N, 16), jnp.float32),
            pltpu.SemaphoreType.DMA,
        ],
    )(_pre_kernel)
    R2 = pre(pos16, src, dst, e_mask, z16)

    ed2, ptn = pl.pallas_call(
        _prep_body,
        grid=(NB,),
        in_specs=[
            pl.BlockSpec((BN, 16), lambda i: (i, 0)),
            pl.BlockSpec((BN, 16), lambda i: (NB + i, 0)),
            pl.BlockSpec((BN, 9), lambda i: (i, 0)),
            pl.BlockSpec((3, D), lambda i: (0, 0)),
            pl.BlockSpec((1, D), lambda i: (0, 0)),
        ],
        out_specs=[
            pl.BlockSpec((BN, 16), lambda i: (i, 0)),
            pl.BlockSpec((BN, D), lambda i: (i, 0)),
        ],
        out_shape=[
            jax.ShapeDtypeStruct((N, 16), jnp.float32),
            jax.ShapeDtypeStruct((N, D), jnp.float32),
        ],
    )(R2, R2, v, W3, b_msg.reshape(1, D))

    spmm = functools.partial(
        pl.kernel,
        mesh=_sc_mesh(),
        compiler_params=pltpu.CompilerParams(use_tc_tiling_on_sc=False),
        out_type=jax.ShapeDtypeStruct((2 * N, 32), jnp.float32),
        scratch_types=[
            pltpu.VMEM((WE,), jnp.int32),
            pltpu.VMEM((WE,), jnp.int32),
            pltpu.VMEM((WE,), jnp.float32),
            pltpu.VMEM((WE, 32), jnp.float32),
            pltpu.VMEM_SHARED((N, 32), jnp.float32),
            pltpu.SemaphoreType.DMA,
        ],
    )(_spmm_kernel)

    post = pl.pallas_call(
        _post_body,
        grid=(NB,),
        in_specs=[
            pl.BlockSpec((BN, D), lambda i: (i, 0)),
            pl.BlockSpec((BN, 32), lambda i: (i, 0)),
            pl.BlockSpec((BN, 32), lambda i: (NB + i, 0)),
            pl.BlockSpec((BN, 16), lambda i: (i, 0)),
            pl.BlockSpec((BN, D), lambda i: (i, 0)),
            pl.BlockSpec((D, D), lambda i: (0, 0)),
            pl.BlockSpec((D, D), lambda i: (0, 0)),
            pl.BlockSpec((D, D), lambda i: (0, 0)),
            pl.BlockSpec((1, D), lambda i: (0, 0)),
            pl.BlockSpec((D, D), lambda i: (0, 0)),
        ],
        out_specs=[
            pl.BlockSpec((BN, D), lambda i: (i, 0)),
            pl.BlockSpec((BN, 32), lambda i: (i, 0)),
            pl.BlockSpec((BN, 32), lambda i: (i, 0)),
        ],
        out_shape=[
            jax.ShapeDtypeStruct((N, D), jnp.float32),
            jax.ShapeDtypeStruct((N, 32), jnp.float32),
            jax.ShapeDtypeStruct((N, 32), jnp.float32),
        ],
    )

    for _ in range(3):
        S2 = spmm(ylo, yhi, src, dst, e_mask, z32)
        x, ylo, yhi = post(x, S2, S2, ed2, ptn, W1, Wu1, Wu2,
                           b_upd.reshape(1, D), W2)
    return x


# speedup vs baseline: 6.6947x; 1.5487x over previous
import functools

import jax
import jax.numpy as jnp
import numpy as np
from jax import lax
from jax.experimental import pallas as pl
from jax.experimental.pallas import tpu as pltpu
from jax.experimental.pallas import tpu_sc as plsc

N = 50000
E = 800000
H = 32
D = 64
B = 16

BN = 400
NB = N // BN
WE = 128
NT = 16
CH = 3128
CH_LAST = N - (NT - 1) * CH


def _pe_table(d_model=32, max_len=20):
    pe = np.zeros((max_len, d_model), dtype=np.float32)
    position = np.arange(0, max_len, dtype=np.float32)[:, None]
    div_term = np.exp(np.arange(0, d_model, 2, dtype=np.float32) * (-np.log(10000.0) / d_model))
    pe[:, 0::2] = np.sin(position * div_term)
    pe[:, 1::2] = np.cos(position * div_term)
    return pe


_PE = _pe_table()


def _pool_body(vfc_ref, vbi_ref, out_ref):
    @pl.when(pl.program_id(0) == 0)
    def _():
        out_ref[...] = jnp.full((8, B), 127, jnp.int32)

    vfc_b = vfc_ref[0, 0, :]
    vbi_b = vbi_ref[0, 0, :]
    oh = vbi_b[:, None] == lax.broadcasted_iota(jnp.int32, (BN, B), 1)
    masked = jnp.where(oh, vfc_b[:, None], 127)
    colmin = jnp.min(masked, axis=0)
    out_ref[...] = jnp.minimum(out_ref[...], colmin[None, :])


def _enc_body(v_ref, l_ref, vfc_ref, vbi_ref, pool_ref, pe_ref,
              wf_ref, bf_ref, wl_ref, bl_ref, w2_ref,
              x_ref, pos16_ref, ylo_ref, yhi_ref):
    v_blk = v_ref[...]
    nonpos = jnp.concatenate([v_blk[:, 0:3], v_blk[:, 6:9]], axis=1)
    h = jnp.dot(nonpos, wf_ref[...], preferred_element_type=jnp.float32) + bf_ref[...]
    vfc_b = vfc_ref[0, 0, :]
    vbi_b = vbi_ref[0, 0, :]
    oh16 = vbi_b[:, None] == lax.broadcasted_iota(jnp.int32, (BN, B), 1)
    poolg = jnp.sum(jnp.where(oh16, pool_ref[0:1, :], 0), axis=1)
    lvl = vfc_b - poolg
    oh20 = (lvl[:, None] == lax.broadcasted_iota(jnp.int32, (BN, 20), 1)).astype(jnp.float32)
    pe_add = jnp.dot(oh20, pe_ref[...], preferred_element_type=jnp.float32)
    le = jnp.dot(l_ref[...], wl_ref[...], preferred_element_type=jnp.float32) + bl_ref[...]
    x = jnp.concatenate([h + pe_add, le], axis=1)
    x_ref[...] = x
    y = jnp.dot(x, w2_ref[...], preferred_element_type=jnp.float32)
    ylo_ref[...] = y[:, :32]
    yhi_ref[...] = y[:, 32:]
    pos = v_blk[:, 3:6]
    ones = jnp.ones((BN, 2), jnp.float32)
    pos16_ref[...] = jnp.concatenate([pos, ones, jnp.zeros((BN, 11), jnp.float32)], axis=1)


def _prep_body(r0_ref, r1_ref, v_ref, w3_ref, bm_ref, ed2_ref, ptn_ref):
    Rr = r0_ref[...] + r1_ref[...]
    Apos = Rr[:, 0:3]
    em = Rr[:, 3:4]
    degc = Rr[:, 4:5]
    invdeg = 1.0 / jnp.maximum(degc, 1.0)
    emn = em * invdeg
    pos = v_ref[...][:, 3:6]
    ptn = (jnp.dot(pos * em - Apos, w3_ref[...], preferred_element_type=jnp.float32)
           + em * bm_ref[...]) * invdeg
    ptn_ref[...] = ptn
    ed2_ref[...] = jnp.concatenate([emn, invdeg, jnp.zeros((BN, 14), jnp.float32)], axis=1)


def _post_body(x_ref, s0_ref, s1_ref, ed2_ref, ptn_ref,
               w1_ref, wu1_ref, wu2_ref, bu_ref, w2_ref,
               xn_ref, ylo_ref, yhi_ref):
    x = x_ref[...]
    S = jnp.concatenate([s0_ref[...], s1_ref[...]], axis=1)
    emn = ed2_ref[...][:, 0:1]
    invdeg = ed2_ref[...][:, 1:2]
    aggr = (jnp.dot(x, w1_ref[...], preferred_element_type=jnp.float32) * emn
            + S * invdeg + ptn_ref[...])
    upd = (jnp.dot(x, wu1_ref[...], preferred_element_type=jnp.float32)
           + jnp.dot(aggr, wu2_ref[...], preferred_element_type=jnp.float32)
           + bu_ref[...])
    xn = x + upd
    xn_ref[...] = xn
    y = jnp.dot(xn, w2_ref[...], preferred_element_type=jnp.float32)
    ylo_ref[...] = y[:, :32]
    yhi_ref[...] = y[:, 32:]


def _sc_mesh():
    return plsc.VectorSubcoreMesh(core_axis_name="c", subcore_axis_name="s")


def _copy_tile_rows(w, src_at, dst_at):
    @pl.when(w < NT - 1)
    def _():
        start = pl.multiple_of(w * CH, 8)
        pltpu.sync_copy(src_at(start, CH), dst_at(start, CH))

    @pl.when(w == NT - 1)
    def _():
        start = (NT - 1) * CH
        pltpu.sync_copy(src_at(start, CH_LAST), dst_at(start, CH_LAST))


def _scale_rows(rows_v, m_v, ncols, masked16=False):
    def grp(g, carry):
        m16 = m_v[pl.ds(g * 16, 16)]
        for e in range(16):
            eix = g * 16 + e
            sv = jnp.full((16,), 1.0, jnp.float32) * m16[e]
            if masked16:
                keep = lax.iota(jnp.int32, 16) < 4
                sv = jnp.where(keep, sv, 1.0)
                rows_v[eix, pl.ds(0, 16)] = rows_v[eix, pl.ds(0, 16)] * sv
            else:
                for f0 in range(0, ncols, 16):
                    rows_v[eix, pl.ds(f0, 16)] = rows_v[eix, pl.ds(f0, 16)] * sv
        return carry
    lax.fori_loop(0, WE // 16, grp, 0)


def _edge_pipeline(c, w, pk_hbm, m_hbm, tables, ncols, masked16, acc,
                   pks, mvs, rows, gsems, nwin, win_off):
    def gather_rows(b):
        for ci, tab in enumerate(tables):
            if len(tables) == 1:
                pltpu.make_async_copy(tab.at[pks[b].at[0]], rows[b],
                                      gsems[b]).start()
            else:
                @pl.when(c == ci)
                def _():
                    pltpu.make_async_copy(tab.at[pks[b].at[0]], rows[b],
                                          gsems[b]).start()

    def wait_rows(b):
        for ci, tab in enumerate(tables):
            if len(tables) == 1:
                pltpu.make_async_copy(tab.at[pks[b].at[0]], rows[b],
                                      gsems[b]).wait()
            else:
                @pl.when(c == ci)
                def _():
                    pltpu.make_async_copy(tab.at[pks[b].at[0]], rows[b],
                                          gsems[b]).wait()

    def stage(b, jj):
        @pl.when(w + NT * jj < nwin)
        def _():
            widx = win_off + w + NT * jj
            pltpu.sync_copy(pk_hbm.at[widx], pks[b])
            pltpu.sync_copy(m_hbm.at[widx], mvs[b])
            gather_rows(b)

    def consume(b, jj):
        @pl.when(w + NT * jj < nwin)
        def _():
            wait_rows(b)
            _scale_rows(rows[b], mvs[b], ncols, masked16=masked16)
            pltpu.sync_copy(rows[b], acc.at[pks[b].at[1]], add=True)

    stage(0, jnp.int32(0))

    def pair(j2, carry):
        for b in (0, 1):
            jj = 2 * j2 + b
            stage(1 - b, jj + 1)
            consume(b, jj)
        return carry

    npairs = ((nwin + NT - 1) // NT + 1) // 2
    lax.fori_loop(0, npairs, pair, 0)


def _spmm_kernel(ylo_hbm, yhi_hbm, pk_hbm, m_hbm, z_hbm, out_hbm,
                 pk0, pk1, mv0, mv1, r0, r1, acc, g0, g1):
    c = lax.axis_index("c")
    w = lax.axis_index("s")
    _copy_tile_rows(w, lambda s, n: z_hbm.at[pl.ds(s, n)],
                    lambda s, n: acc.at[pl.ds(s, n)])
    plsc.subcore_barrier()
    _edge_pipeline(c, w, pk_hbm, m_hbm, [ylo_hbm, yhi_hbm], 32, False, acc,
                   [pk0, pk1], [mv0, mv1], [r0, r1], [g0, g1], E // WE, 0)
    plsc.subcore_barrier()
    _copy_tile_rows(w, lambda s, n: acc.at[pl.ds(s, n)],
                    lambda s, n: out_hbm.at[pl.ds(c * N + s, n)])


def _pre_kernel(tab_hbm, pk_hbm, m_hbm, z_hbm, out_hbm,
                pk0, pk1, mv0, mv1, r0, r1, acc, g0, g1):
    c = lax.axis_index("c")
    w = lax.axis_index("s")
    _copy_tile_rows(w, lambda s, n: z_hbm.at[pl.ds(s, n)],
                    lambda s, n: acc.at[pl.ds(s, n)])
    plsc.subcore_barrier()
    nwin_half = E // (2 * WE)
    _edge_pipeline(c, w, pk_hbm, m_hbm, [tab_hbm], 16, True, acc,
                   [pk0, pk1], [mv0, mv1], [r0, r1], [g0, g1],
                   nwin_half, c * nwin_half)
    plsc.subcore_barrier()
    _copy_tile_rows(w, lambda s, n: acc.at[pl.ds(s, n)],
                    lambda s, n: out_hbm.at[pl.ds(c * N + s, n)])


@functools.partial(jax.jit, static_argnums=())
def kernel(v, l, e, e_mask, vbi, vfc, vfb, W_feat, b_feat, W_lab, b_lab,
           W_msg, b_msg, W_upd, b_upd):
    src = e[0]
    dst = e[1]
    W1 = W_msg[0:D]
    W2 = W_msg[D:2 * D]
    W3 = W_msg[2 * D:]
    Wu1 = W_upd[0:D]
    Wu2 = W_upd[D:]
    pe = jnp.asarray(_PE)
    vfc3 = vfc.reshape(NB, 1, BN)
    vbi3 = vbi.reshape(NB, 1, BN)

    pool8 = pl.pallas_call(
        _pool_body,
        grid=(NB,),
        in_specs=[
            pl.BlockSpec((1, 1, BN), lambda i: (i, 0, 0)),
            pl.BlockSpec((1, 1, BN), lambda i: (i, 0, 0)),
        ],
        out_specs=pl.BlockSpec((8, B), lambda i: (0, 0)),
        out_shape=jax.ShapeDtypeStruct((8, B), jnp.int32),
    )(vfc3, vbi3)

    x, pos16, ylo, yhi = pl.pallas_call(
        _enc_body,
        grid=(NB,),
        in_specs=[
            pl.BlockSpec((BN, 9), lambda i: (i, 0)),
            pl.BlockSpec((BN, 8), lambda i: (i, 0)),
            pl.BlockSpec((1, 1, BN), lambda i: (i, 0, 0)),
            pl.BlockSpec((1, 1, BN), lambda i: (i, 0, 0)),
            pl.BlockSpec((8, B), lambda i: (0, 0)),
            pl.BlockSpec((20, 32), lambda i: (0, 0)),
            pl.BlockSpec((6, 32), lambda i: (0, 0)),
            pl.BlockSpec((1, 32), lambda i: (0, 0)),
            pl.BlockSpec((8, 32), lambda i: (0, 0)),
            pl.BlockSpec((1, 32), lambda i: (0, 0)),
            pl.BlockSpec((D, D), lambda i: (0, 0)),
        ],
        out_specs=[
            pl.BlockSpec((BN, D), lambda i: (i, 0)),
            pl.BlockSpec((BN, 16), lambda i: (i, 0)),
            pl.BlockSpec((BN, 32), lambda i: (i, 0)),
            pl.BlockSpec((BN, 32), lambda i: (i, 0)),
        ],
        out_shape=[
            jax.ShapeDtypeStruct((N, D), jnp.float32),
            jax.ShapeDtypeStruct((N, 16), jnp.float32),
            jax.ShapeDtypeStruct((N, 32), jnp.float32),
            jax.ShapeDtypeStruct((N, 32), jnp.float32),
        ],
    )(v, l, vfc3, vbi3, pool8, pe, W_feat, b_feat.reshape(1, 32),
      W_lab, b_lab.reshape(1, 32), W2)

    z16 = jnp.zeros((N, 16), jnp.float32)
    z32 = jnp.zeros((N, 32), jnp.float32)
    NW = E // WE
    pk = jnp.concatenate([
        src.reshape(NW, 1, WE),
        dst.reshape(NW, 1, WE),
    ], axis=1)
    mw = e_mask.reshape(NW, WE)

    pre = functools.partial(
        pl.kernel,
        mesh=_sc_mesh(),
        compiler_params=pltpu.CompilerParams(use_tc_tiling_on_sc=False),
        out_type=jax.ShapeDtypeStruct((2 * N, 16), jnp.float32),
        scratch_types=[
            pltpu.VMEM((2, WE), jnp.int32),
            pltpu.VMEM((2, WE), jnp.int32),
            pltpu.VMEM((WE,), jnp.float32),
            pltpu.VMEM((WE,), jnp.float32),
            pltpu.VMEM((WE, 16), jnp.float32),
            pltpu.VMEM((WE, 16), jnp.float32),
            pltpu.VMEM_SHARED((N, 16), jnp.float32),
            pltpu.SemaphoreType.DMA,
            pltpu.SemaphoreType.DMA,
        ],
    )(_pre_kernel)
    R2 = pre(pos16, pk, mw, z16)

    ed2, ptn = pl.pallas_call(
        _prep_body,
        grid=(NB,),
        in_specs=[
            pl.BlockSpec((BN, 16), lambda i: (i, 0)),
            pl.BlockSpec((BN, 16), lambda i: (NB + i, 0)),
            pl.BlockSpec((BN, 9), lambda i: (i, 0)),
            pl.BlockSpec((3, D), lambda i: (0, 0)),
            pl.BlockSpec((1, D), lambda i: (0, 0)),
        ],
        out_specs=[
            pl.BlockSpec((BN, 16), lambda i: (i, 0)),
            pl.BlockSpec((BN, D), lambda i: (i, 0)),
        ],
        out_shape=[
            jax.ShapeDtypeStruct((N, 16), jnp.float32),
            jax.ShapeDtypeStruct((N, D), jnp.float32),
        ],
    )(R2, R2, v, W3, b_msg.reshape(1, D))

    spmm = functools.partial(
        pl.kernel,
        mesh=_sc_mesh(),
        compiler_params=pltpu.CompilerParams(use_tc_tiling_on_sc=False),
        out_type=jax.ShapeDtypeStruct((2 * N, 32), jnp.float32),
        scratch_types=[
            pltpu.VMEM((2, WE), jnp.int32),
            pltpu.VMEM((2, WE), jnp.int32),
            pltpu.VMEM((WE,), jnp.float32),
            pltpu.VMEM((WE,), jnp.float32),
            pltpu.VMEM((WE, 32), jnp.float32),
            pltpu.VMEM((WE, 32), jnp.float32),
            pltpu.VMEM_SHARED((N, 32), jnp.float32),
            pltpu.SemaphoreType.DMA,
            pltpu.SemaphoreType.DMA,
        ],
    )(_spmm_kernel)

    post = pl.pallas_call(
        _post_body,
        grid=(NB,),
        in_specs=[
            pl.BlockSpec((BN, D), lambda i: (i, 0)),
            pl.BlockSpec((BN, 32), lambda i: (i, 0)),
            pl.BlockSpec((BN, 32), lambda i: (NB + i, 0)),
            pl.BlockSpec((BN, 16), lambda i: (i, 0)),
            pl.BlockSpec((BN, D), lambda i: (i, 0)),
            pl.BlockSpec((D, D), lambda i: (0, 0)),
            pl.BlockSpec((D, D), lambda i: (0, 0)),
            pl.BlockSpec((D, D), lambda i: (0, 0)),
            pl.BlockSpec((1, D), lambda i: (0, 0)),
            pl.BlockSpec((D, D), lambda i: (0, 0)),
        ],
        out_specs=[
            pl.BlockSpec((BN, D), lambda i: (i, 0)),
            pl.BlockSpec((BN, 32), lambda i: (i, 0)),
            pl.BlockSpec((BN, 32), lambda i: (i, 0)),
        ],
        out_shape=[
            jax.ShapeDtypeStruct((N, D), jnp.float32),
            jax.ShapeDtypeStruct((N, 32), jnp.float32),
            jax.ShapeDtypeStruct((N, 32), jnp.float32),
        ],
    )

    for _ in range(3):
        S2 = spmm(ylo, yhi, pk, mw, z32)
        x, ylo, yhi = post(x, S2, S2, ed2, ptn, W1, Wu1, Wu2,
                           b_upd.reshape(1, D), W2)
    return x


# speedup vs baseline: 8.6272x; 1.2887x over previous
import functools

import jax
import jax.numpy as jnp
import numpy as np
from jax import lax
from jax.experimental import pallas as pl
from jax.experimental.pallas import tpu as pltpu
from jax.experimental.pallas import tpu_sc as plsc

N = 50000
E = 800000
H = 32
D = 64
B = 16

BN = 400
NB = N // BN
WE = 128
NT = 16
CH = 3128
CH_LAST = N - (NT - 1) * CH


def _pe_table(d_model=32, max_len=20):
    pe = np.zeros((max_len, d_model), dtype=np.float32)
    position = np.arange(0, max_len, dtype=np.float32)[:, None]
    div_term = np.exp(np.arange(0, d_model, 2, dtype=np.float32) * (-np.log(10000.0) / d_model))
    pe[:, 0::2] = np.sin(position * div_term)
    pe[:, 1::2] = np.cos(position * div_term)
    return pe


_PE = _pe_table()


def _pool_body(vfc_ref, vbi_ref, out_ref):
    @pl.when(pl.program_id(0) == 0)
    def _():
        out_ref[...] = jnp.full((8, B), 127, jnp.int32)

    vfc_b = vfc_ref[0, 0, :]
    vbi_b = vbi_ref[0, 0, :]
    oh = vbi_b[:, None] == lax.broadcasted_iota(jnp.int32, (BN, B), 1)
    masked = jnp.where(oh, vfc_b[:, None], 127)
    colmin = jnp.min(masked, axis=0)
    out_ref[...] = jnp.minimum(out_ref[...], colmin[None, :])


def _enc_body(v_ref, l_ref, vfc_ref, vbi_ref, pool_ref, pe_ref,
              wf_ref, bf_ref, wl_ref, bl_ref, w2_ref,
              x_ref, pos16_ref, ylo_ref, yhi_ref):
    v_blk = v_ref[...]
    nonpos = jnp.concatenate([v_blk[:, 0:3], v_blk[:, 6:9]], axis=1)
    h = jnp.dot(nonpos, wf_ref[...], preferred_element_type=jnp.float32) + bf_ref[...]
    vfc_b = vfc_ref[0, 0, :]
    vbi_b = vbi_ref[0, 0, :]
    oh16 = vbi_b[:, None] == lax.broadcasted_iota(jnp.int32, (BN, B), 1)
    poolg = jnp.sum(jnp.where(oh16, pool_ref[0:1, :], 0), axis=1)
    lvl = vfc_b - poolg
    oh20 = (lvl[:, None] == lax.broadcasted_iota(jnp.int32, (BN, 20), 1)).astype(jnp.float32)
    pe_add = jnp.dot(oh20, pe_ref[...], preferred_element_type=jnp.float32)
    le = jnp.dot(l_ref[...], wl_ref[...], preferred_element_type=jnp.float32) + bl_ref[...]
    x = jnp.concatenate([h + pe_add, le], axis=1)
    x_ref[...] = x
    y = jnp.dot(x, w2_ref[...], preferred_element_type=jnp.float32)
    ylo_ref[...] = y[:, :32]
    yhi_ref[...] = y[:, 32:]
    pos = v_blk[:, 3:6]
    ones = jnp.ones((BN, 2), jnp.float32)
    pos16_ref[...] = jnp.concatenate([pos, ones, jnp.zeros((BN, 11), jnp.float32)], axis=1)


def _prep_body(r0_ref, r1_ref, v_ref, w3_ref, bm_ref, ed2_ref, ptn_ref):
    Rr = r0_ref[...] + r1_ref[...]
    Apos = Rr[:, 0:3]
    em = Rr[:, 3:4]
    degc = Rr[:, 4:5]
    invdeg = 1.0 / jnp.maximum(degc, 1.0)
    emn = em * invdeg
    pos = v_ref[...][:, 3:6]
    ptn = (jnp.dot(pos * em - Apos, w3_ref[...], preferred_element_type=jnp.float32)
           + em * bm_ref[...]) * invdeg
    ptn_ref[...] = ptn
    ed2_ref[...] = jnp.concatenate([emn, invdeg, jnp.zeros((BN, 14), jnp.float32)], axis=1)


def _post_body(x_ref, s0_ref, s1_ref, ed2_ref, ptn_ref,
               w1_ref, wu1_ref, wu2_ref, bu_ref, w2_ref,
               xn_ref, ylo_ref, yhi_ref):
    x = x_ref[...]
    S = jnp.concatenate([s0_ref[...], s1_ref[...]], axis=1)
    emn = ed2_ref[...][:, 0:1]
    invdeg = ed2_ref[...][:, 1:2]
    aggr = (jnp.dot(x, w1_ref[...], preferred_element_type=jnp.float32) * emn
            + S * invdeg + ptn_ref[...])
    upd = (jnp.dot(x, wu1_ref[...], preferred_element_type=jnp.float32)
           + jnp.dot(aggr, wu2_ref[...], preferred_element_type=jnp.float32)
           + bu_ref[...])
    xn = x + upd
    xn_ref[...] = xn
    y = jnp.dot(xn, w2_ref[...], preferred_element_type=jnp.float32)
    ylo_ref[...] = y[:, :32]
    yhi_ref[...] = y[:, 32:]


def _sc_mesh():
    return plsc.VectorSubcoreMesh(core_axis_name="c", subcore_axis_name="s")


def _copy_tile_rows(w, src_at, dst_at):
    @pl.when(w < NT - 1)
    def _():
        start = pl.multiple_of(w * CH, 8)
        pltpu.sync_copy(src_at(start, CH), dst_at(start, CH))

    @pl.when(w == NT - 1)
    def _():
        start = (NT - 1) * CH
        pltpu.sync_copy(src_at(start, CH_LAST), dst_at(start, CH_LAST))


def _scale_rows(rows_v, m_v, ncols, masked16=False):
    def grp(g, carry):
        m16 = m_v[pl.ds(g * 16, 16)]
        for e in range(16):
            eix = g * 16 + e
            sv = jnp.full((16,), 1.0, jnp.float32) * m16[e]
            if masked16:
                keep = lax.iota(jnp.int32, 16) < 4
                sv = jnp.where(keep, sv, 1.0)
                rows_v[eix, pl.ds(0, 16)] = rows_v[eix, pl.ds(0, 16)] * sv
            else:
                for f0 in range(0, ncols, 16):
                    rows_v[eix, pl.ds(f0, 16)] = rows_v[eix, pl.ds(f0, 16)] * sv
        return carry
    lax.fori_loop(0, 2 * WE // 16, grp, 0)


def _edge_pipeline(c, w, pk_hbm, m_hbm, tables, ncols, masked16, acc,
                   pks, mvs, rows, gsems, ssems, nwin, win_off, npairs):
    def on_table(b, fn):
        for ci, tab in enumerate(tables):
            if len(tables) == 1:
                fn(tab, b)
            else:
                @pl.when(c == ci)
                def _():
                    fn(tab, b)

    def gather_rows(tab, b):
        pltpu.make_async_copy(tab.at[pks[b].at[0]],
                              rows[b].at[pl.ds(0, WE)], gsems[b]).start()
        pltpu.make_async_copy(tab.at[pks[b].at[1]],
                              rows[b].at[pl.ds(WE, WE)], gsems[b]).start()

    def wait_rows(tab, b):
        pltpu.make_async_copy(tab.at[pks[b].at[0]],
                              rows[b].at[pl.ds(0, WE)], gsems[b]).wait()
        pltpu.make_async_copy(tab.at[pks[b].at[1]],
                              rows[b].at[pl.ds(WE, WE)], gsems[b]).wait()

    def wait_scatter(b):
        pltpu.make_async_copy(rows[b].at[pl.ds(0, WE)],
                              acc.at[pks[b].at[2]], ssems[b]).wait()
        pltpu.make_async_copy(rows[b].at[pl.ds(WE, WE)],
                              acc.at[pks[b].at[3]], ssems[b]).wait()

    def stage(b, jjt):
        @pl.when(w + NT * jjt < nwin)
        def _():
            @pl.when(jjt >= 2)
            def _():
                wait_scatter(b)
            widx = win_off + w + NT * jjt
            pltpu.sync_copy(pk_hbm.at[widx], pks[b])
            pltpu.sync_copy(m_hbm.at[widx], mvs[b])
            on_table(b, gather_rows)

    def consume(b, jj):
        @pl.when(w + NT * jj < nwin)
        def _():
            on_table(b, wait_rows)
            _scale_rows(rows[b], mvs[b], ncols, masked16=masked16)
            pltpu.async_copy(rows[b].at[pl.ds(0, WE)],
                             acc.at[pks[b].at[2]], ssems[b], add=True)
            pltpu.async_copy(rows[b].at[pl.ds(WE, WE)],
                             acc.at[pks[b].at[3]], ssems[b], add=True)

    stage(0, jnp.int32(0))

    def pair(j2, carry):
        for b in (0, 1):
            jj = 2 * j2 + b
            stage(1 - b, jj + 1)
            consume(b, jj)
        return carry

    lax.fori_loop(0, npairs, pair, 0)

    for jj in range(max(0, 2 * npairs - 4), 2 * npairs):
        @pl.when((w + NT * jj < nwin) & (w + NT * (jj + 2) >= nwin))
        def _():
            wait_scatter(jj % 2)


NWIN = E // (2 * WE)
NPAIRS_SPMM = ((NWIN + NT - 1) // NT + 1) // 2
PRE_SPLIT = (NWIN + 1) // 2
NPAIRS_PRE = ((PRE_SPLIT + NT - 1) // NT + 1) // 2


def _spmm_kernel(ylo_hbm, yhi_hbm, pk_hbm, m_hbm, z_hbm, out_hbm,
                 pk0, pk1, mv0, mv1, r0, r1, acc, g0, g1, s0, s1):
    c = lax.axis_index("c")
    w = lax.axis_index("s")
    _copy_tile_rows(w, lambda s, n: z_hbm.at[pl.ds(s, n)],
                    lambda s, n: acc.at[pl.ds(s, n)])
    plsc.subcore_barrier()
    _edge_pipeline(c, w, pk_hbm, m_hbm, [ylo_hbm, yhi_hbm], 32, False, acc,
                   [pk0, pk1], [mv0, mv1], [r0, r1], [g0, g1], [s0, s1],
                   NWIN, 0, NPAIRS_SPMM)
    plsc.subcore_barrier()
    _copy_tile_rows(w, lambda s, n: acc.at[pl.ds(s, n)],
                    lambda s, n: out_hbm.at[pl.ds(c * N + s, n)])


def _pre_kernel(tab_hbm, pk_hbm, m_hbm, z_hbm, out_hbm,
                pk0, pk1, mv0, mv1, r0, r1, acc, g0, g1, s0, s1):
    c = lax.axis_index("c")
    w = lax.axis_index("s")
    _copy_tile_rows(w, lambda s, n: z_hbm.at[pl.ds(s, n)],
                    lambda s, n: acc.at[pl.ds(s, n)])
    plsc.subcore_barrier()
    _edge_pipeline(c, w, pk_hbm, m_hbm, [tab_hbm], 16, True, acc,
                   [pk0, pk1], [mv0, mv1], [r0, r1], [g0, g1], [s0, s1],
                   PRE_SPLIT - c, c * PRE_SPLIT, NPAIRS_PRE)
    plsc.subcore_barrier()
    _copy_tile_rows(w, lambda s, n: acc.at[pl.ds(s, n)],
                    lambda s, n: out_hbm.at[pl.ds(c * N + s, n)])


@functools.partial(jax.jit, static_argnums=())
def kernel(v, l, e, e_mask, vbi, vfc, vfb, W_feat, b_feat, W_lab, b_lab,
           W_msg, b_msg, W_upd, b_upd):
    src = e[0]
    dst = e[1]
    W1 = W_msg[0:D]
    W2 = W_msg[D:2 * D]
    W3 = W_msg[2 * D:]
    Wu1 = W_upd[0:D]
    Wu2 = W_upd[D:]
    pe = jnp.asarray(_PE)
    vfc3 = vfc.reshape(NB, 1, BN)
    vbi3 = vbi.reshape(NB, 1, BN)

    pool8 = pl.pallas_call(
        _pool_body,
        grid=(NB,),
        in_specs=[
            pl.BlockSpec((1, 1, BN), lambda i: (i, 0, 0)),
            pl.BlockSpec((1, 1, BN), lambda i: (i, 0, 0)),
        ],
        out_specs=pl.BlockSpec((8, B), lambda i: (0, 0)),
        out_shape=jax.ShapeDtypeStruct((8, B), jnp.int32),
    )(vfc3, vbi3)

    x, pos16, ylo, yhi = pl.pallas_call(
        _enc_body,
        grid=(NB,),
        in_specs=[
            pl.BlockSpec((BN, 9), lambda i: (i, 0)),
            pl.BlockSpec((BN, 8), lambda i: (i, 0)),
            pl.BlockSpec((1, 1, BN), lambda i: (i, 0, 0)),
            pl.BlockSpec((1, 1, BN), lambda i: (i, 0, 0)),
            pl.BlockSpec((8, B), lambda i: (0, 0)),
            pl.BlockSpec((20, 32), lambda i: (0, 0)),
            pl.BlockSpec((6, 32), lambda i: (0, 0)),
            pl.BlockSpec((1, 32), lambda i: (0, 0)),
            pl.BlockSpec((8, 32), lambda i: (0, 0)),
            pl.BlockSpec((1, 32), lambda i: (0, 0)),
            pl.BlockSpec((D, D), lambda i: (0, 0)),
        ],
        out_specs=[
            pl.BlockSpec((BN, D), lambda i: (i, 0)),
            pl.BlockSpec((BN, 16), lambda i: (i, 0)),
            pl.BlockSpec((BN, 32), lambda i: (i, 0)),
            pl.BlockSpec((BN, 32), lambda i: (i, 0)),
        ],
        out_shape=[
            jax.ShapeDtypeStruct((N, D), jnp.float32),
            jax.ShapeDtypeStruct((N, 16), jnp.float32),
            jax.ShapeDtypeStruct((N, 32), jnp.float32),
            jax.ShapeDtypeStruct((N, 32), jnp.float32),
        ],
    )(v, l, vfc3, vbi3, pool8, pe, W_feat, b_feat.reshape(1, 32),
      W_lab, b_lab.reshape(1, 32), W2)

    z16 = jnp.zeros((N, 16), jnp.float32)
    z32 = jnp.zeros((N, 32), jnp.float32)
    pk = jnp.concatenate([
        src.reshape(NWIN, 2, WE),
        dst.reshape(NWIN, 2, WE),
    ], axis=1)
    mw = e_mask.reshape(NWIN, 2 * WE)

    pre = functools.partial(
        pl.kernel,
        mesh=_sc_mesh(),
        compiler_params=pltpu.CompilerParams(use_tc_tiling_on_sc=False),
        out_type=jax.ShapeDtypeStruct((2 * N, 16), jnp.float32),
        scratch_types=[
            pltpu.VMEM((4, WE), jnp.int32),
            pltpu.VMEM((4, WE), jnp.int32),
            pltpu.VMEM((2 * WE,), jnp.float32),
            pltpu.VMEM((2 * WE,), jnp.float32),
            pltpu.VMEM((2 * WE, 16), jnp.float32),
            pltpu.VMEM((2 * WE, 16), jnp.float32),
            pltpu.VMEM_SHARED((N, 16), jnp.float32),
            pltpu.SemaphoreType.DMA,
            pltpu.SemaphoreType.DMA,
            pltpu.SemaphoreType.DMA,
            pltpu.SemaphoreType.DMA,
        ],
    )(_pre_kernel)
    R2 = pre(pos16, pk, mw, z16)

    ed2, ptn = pl.pallas_call(
        _prep_body,
        grid=(NB,),
        in_specs=[
            pl.BlockSpec((BN, 16), lambda i: (i, 0)),
            pl.BlockSpec((BN, 16), lambda i: (NB + i, 0)),
            pl.BlockSpec((BN, 9), lambda i: (i, 0)),
            pl.BlockSpec((3, D), lambda i: (0, 0)),
            pl.BlockSpec((1, D), lambda i: (0, 0)),
        ],
        out_specs=[
            pl.BlockSpec((BN, 16), lambda i: (i, 0)),
            pl.BlockSpec((BN, D), lambda i: (i, 0)),
        ],
        out_shape=[
            jax.ShapeDtypeStruct((N, 16), jnp.float32),
            jax.ShapeDtypeStruct((N, D), jnp.float32),
        ],
    )(R2, R2, v, W3, b_msg.reshape(1, D))

    spmm = functools.partial(
        pl.kernel,
        mesh=_sc_mesh(),
        compiler_params=pltpu.CompilerParams(use_tc_tiling_on_sc=False),
        out_type=jax.ShapeDtypeStruct((2 * N, 32), jnp.float32),
        scratch_types=[
            pltpu.VMEM((4, WE), jnp.int32),
            pltpu.VMEM((4, WE), jnp.int32),
            pltpu.VMEM((2 * WE,), jnp.float32),
            pltpu.VMEM((2 * WE,), jnp.float32),
            pltpu.VMEM((2 * WE, 32), jnp.float32),
            pltpu.VMEM((2 * WE, 32), jnp.float32),
            pltpu.VMEM_SHARED((N, 32), jnp.float32),
            pltpu.SemaphoreType.DMA,
            pltpu.SemaphoreType.DMA,
            pltpu.SemaphoreType.DMA,
            pltpu.SemaphoreType.DMA,
        ],
    )(_spmm_kernel)

    post = pl.pallas_call(
        _post_body,
        grid=(NB,),
        in_specs=[
            pl.BlockSpec((BN, D), lambda i: (i, 0)),
            pl.BlockSpec((BN, 32), lambda i: (i, 0)),
            pl.BlockSpec((BN, 32), lambda i: (NB + i, 0)),
            pl.BlockSpec((BN, 16), lambda i: (i, 0)),
            pl.BlockSpec((BN, D), lambda i: (i, 0)),
            pl.BlockSpec((D, D), lambda i: (0, 0)),
            pl.BlockSpec((D, D), lambda i: (0, 0)),
            pl.BlockSpec((D, D), lambda i: (0, 0)),
            pl.BlockSpec((1, D), lambda i: (0, 0)),
            pl.BlockSpec((D, D), lambda i: (0, 0)),
        ],
        out_specs=[
            pl.BlockSpec((BN, D), lambda i: (i, 0)),
            pl.BlockSpec((BN, 32), lambda i: (i, 0)),
            pl.BlockSpec((BN, 32), lambda i: (i, 0)),
        ],
        out_shape=[
            jax.ShapeDtypeStruct((N, D), jnp.float32),
            jax.ShapeDtypeStruct((N, 32), jnp.float32),
            jax.ShapeDtypeStruct((N, 32), jnp.float32),
        ],
    )

    for _ in range(3):
        S2 = spmm(ylo, yhi, pk, mw, z32)
        x, ylo, yhi = post(x, S2, S2, ed2, ptn, W1, Wu1, Wu2,
                           b_upd.reshape(1, D), W2)
    return x


# speedup vs baseline: 9.8457x; 1.1412x over previous
import functools

import jax
import jax.numpy as jnp
import numpy as np
from jax import lax
from jax.experimental import pallas as pl
from jax.experimental.pallas import tpu as pltpu
from jax.experimental.pallas import tpu_sc as plsc

N = 50000
E = 800000
H = 32
D = 64
B = 16

BN = 1000
NB = N // BN
WE = 128
KSUB = 2
WWIN = KSUB * WE
NT = 16
CH = 3128
CH_LAST = N - (NT - 1) * CH


def _pe_table(d_model=32, max_len=20):
    pe = np.zeros((max_len, d_model), dtype=np.float32)
    position = np.arange(0, max_len, dtype=np.float32)[:, None]
    div_term = np.exp(np.arange(0, d_model, 2, dtype=np.float32) * (-np.log(10000.0) / d_model))
    pe[:, 0::2] = np.sin(position * div_term)
    pe[:, 1::2] = np.cos(position * div_term)
    return pe


_PE = _pe_table()


def _pool_body(vfc_ref, vbi_ref, out_ref):
    @pl.when(pl.program_id(0) == 0)
    def _():
        out_ref[...] = jnp.full((8, B), 127, jnp.int32)

    vfc_b = vfc_ref[0, 0, :]
    vbi_b = vbi_ref[0, 0, :]
    oh = vbi_b[:, None] == lax.broadcasted_iota(jnp.int32, (BN, B), 1)
    masked = jnp.where(oh, vfc_b[:, None], 127)
    colmin = jnp.min(masked, axis=0)
    out_ref[...] = jnp.minimum(out_ref[...], colmin[None, :])


def _enc_body(v_ref, l_ref, vfc_ref, vbi_ref, pool_ref, pe_ref,
              wf_ref, bf_ref, wl_ref, bl_ref, w2_ref,
              x_ref, pos16_ref, ylo_ref, yhi_ref):
    v_blk = v_ref[...]
    nonpos = jnp.concatenate([v_blk[:, 0:3], v_blk[:, 6:9]], axis=1)
    h = jnp.dot(nonpos, wf_ref[...], preferred_element_type=jnp.float32) + bf_ref[...]
    vfc_b = vfc_ref[0, 0, :]
    vbi_b = vbi_ref[0, 0, :]
    oh16 = vbi_b[:, None] == lax.broadcasted_iota(jnp.int32, (BN, B), 1)
    poolg = jnp.sum(jnp.where(oh16, pool_ref[0:1, :], 0), axis=1)
    lvl = vfc_b - poolg
    oh20 = (lvl[:, None] == lax.broadcasted_iota(jnp.int32, (BN, 20), 1)).astype(jnp.float32)
    pe_add = jnp.dot(oh20, pe_ref[...], preferred_element_type=jnp.float32)
    le = jnp.dot(l_ref[...], wl_ref[...], preferred_element_type=jnp.float32) + bl_ref[...]
    x = jnp.concatenate([h + pe_add, le], axis=1)
    x_ref[...] = x
    y = jnp.dot(x, w2_ref[...], preferred_element_type=jnp.float32)
    ylo_ref[...] = y[:, :32]
    yhi_ref[...] = y[:, 32:]
    pos = v_blk[:, 3:6]
    ones = jnp.ones((BN, 2), jnp.float32)
    pos16_ref[...] = jnp.concatenate([pos, ones, jnp.zeros((BN, 11), jnp.float32)], axis=1)


def _prep_body(r0_ref, r1_ref, v_ref, w3_ref, bm_ref, ed2_ref, ptn_ref):
    Rr = r0_ref[...] + r1_ref[...]
    Apos = Rr[:, 0:3]
    em = Rr[:, 3:4]
    degc = Rr[:, 4:5]
    invdeg = 1.0 / jnp.maximum(degc, 1.0)
    emn = em * invdeg
    pos = v_ref[...][:, 3:6]
    ptn = (jnp.dot(pos * em - Apos, w3_ref[...], preferred_element_type=jnp.float32)
           + em * bm_ref[...]) * invdeg
    ptn_ref[...] = ptn
    ed2_ref[...] = jnp.concatenate([emn, invdeg, jnp.zeros((BN, 14), jnp.float32)], axis=1)


def _post_body(x_ref, s0_ref, s1_ref, ed2_ref, ptn_ref,
               w1_ref, wu1_ref, wu2_ref, bu_ref, w2_ref,
               xn_ref, ylo_ref, yhi_ref):
    x = x_ref[...]
    S = jnp.concatenate([s0_ref[...], s1_ref[...]], axis=1)
    emn = ed2_ref[...][:, 0:1]
    invdeg = ed2_ref[...][:, 1:2]
    aggr = (jnp.dot(x, w1_ref[...], preferred_element_type=jnp.float32) * emn
            + S * invdeg + ptn_ref[...])
    upd = (jnp.dot(x, wu1_ref[...], preferred_element_type=jnp.float32)
           + jnp.dot(aggr, wu2_ref[...], preferred_element_type=jnp.float32)
           + bu_ref[...])
    xn = x + upd
    xn_ref[...] = xn
    y = jnp.dot(xn, w2_ref[...], preferred_element_type=jnp.float32)
    ylo_ref[...] = y[:, :32]
    yhi_ref[...] = y[:, 32:]


def _sc_mesh():
    return plsc.VectorSubcoreMesh(core_axis_name="c", subcore_axis_name="s")


def _copy_tile_rows(w, src_at, dst_at):
    @pl.when(w < NT - 1)
    def _():
        start = pl.multiple_of(w * CH, 8)
        pltpu.sync_copy(src_at(start, CH), dst_at(start, CH))

    @pl.when(w == NT - 1)
    def _():
        start = (NT - 1) * CH
        pltpu.sync_copy(src_at(start, CH_LAST), dst_at(start, CH_LAST))


def _scale_rows(rows_v, m_v, ncols, masked16=False):
    def grp(g, carry):
        m16 = m_v[pl.ds(g * 16, 16)]
        for e in range(16):
            eix = g * 16 + e
            sv = jnp.full((16,), 1.0, jnp.float32) * m16[e]
            if masked16:
                keep = lax.iota(jnp.int32, 16) < 4
                sv = jnp.where(keep, sv, 1.0)
                rows_v[eix, pl.ds(0, 16)] = rows_v[eix, pl.ds(0, 16)] * sv
            else:
                for f0 in range(0, ncols, 16):
                    rows_v[eix, pl.ds(f0, 16)] = rows_v[eix, pl.ds(f0, 16)] * sv
        return carry
    lax.fori_loop(0, WWIN // 16, grp, 0)


def _edge_pipeline(c, w, pk_hbm, m_hbm, tables, ncols, masked16, acc,
                   pks, mvs, rows, gsems, ssems, nwin, win_off, npairs):
    def on_table(b, fn):
        for ci, tab in enumerate(tables):
            if len(tables) == 1:
                fn(tab, b)
            else:
                @pl.when(c == ci)
                def _():
                    fn(tab, b)

    def gather_rows(tab, b):
        for s in range(KSUB):
            pltpu.make_async_copy(tab.at[pks[b].at[s]],
                                  rows[b].at[pl.ds(s * WE, WE)],
                                  gsems[b]).start()

    def wait_rows(tab, b):
        for s in range(KSUB):
            pltpu.make_async_copy(tab.at[pks[b].at[s]],
                                  rows[b].at[pl.ds(s * WE, WE)],
                                  gsems[b]).wait()

    def wait_scatter(b):
        for s in range(KSUB):
            pltpu.make_async_copy(rows[b].at[pl.ds(s * WE, WE)],
                                  acc.at[pks[b].at[KSUB + s]], ssems[b]).wait()

    def stage(b, jjt):
        @pl.when(w + NT * jjt < nwin)
        def _():
            @pl.when(jjt >= 2)
            def _():
                wait_scatter(b)
            widx = win_off + w + NT * jjt
            pltpu.sync_copy(pk_hbm.at[widx], pks[b])
            pltpu.sync_copy(m_hbm.at[widx], mvs[b])
            on_table(b, gather_rows)

    def consume(b, jj):
        @pl.when(w + NT * jj < nwin)
        def _():
            on_table(b, wait_rows)
            _scale_rows(rows[b], mvs[b], ncols, masked16=masked16)
            for s in range(KSUB):
                pltpu.async_copy(rows[b].at[pl.ds(s * WE, WE)],
                                 acc.at[pks[b].at[KSUB + s]], ssems[b],
                                 add=True)

    stage(0, jnp.int32(0))

    def pair(j2, carry):
        for b in (0, 1):
            jj = 2 * j2 + b
            stage(1 - b, jj + 1)
            consume(b, jj)
        return carry

    lax.fori_loop(0, npairs, pair, 0)

    for jj in range(max(0, 2 * npairs - 4), 2 * npairs):
        @pl.when((w + NT * jj < nwin) & (w + NT * (jj + 2) >= nwin))
        def _():
            wait_scatter(jj % 2)


NWIN = E // WWIN
NPAIRS_SPMM = ((NWIN + NT - 1) // NT + 1) // 2
PRE_SPLIT = (NWIN + 1) // 2
NPAIRS_PRE = ((PRE_SPLIT + NT - 1) // NT + 1) // 2


def _spmm_kernel(ylo_hbm, yhi_hbm, pk_hbm, m_hbm, z_hbm, out_hbm,
                 pk0, pk1, mv0, mv1, r0, r1, acc, g0, g1, s0, s1):
    c = lax.axis_index("c")
    w = lax.axis_index("s")
    _copy_tile_rows(w, lambda s, n: z_hbm.at[pl.ds(s, n)],
                    lambda s, n: acc.at[pl.ds(s, n)])
    plsc.subcore_barrier()
    _edge_pipeline(c, w, pk_hbm, m_hbm, [ylo_hbm, yhi_hbm], 32, False, acc,
                   [pk0, pk1], [mv0, mv1], [r0, r1], [g0, g1], [s0, s1],
                   NWIN, 0, NPAIRS_SPMM)
    plsc.subcore_barrier()
    _copy_tile_rows(w, lambda s, n: acc.at[pl.ds(s, n)],
                    lambda s, n: out_hbm.at[pl.ds(c * N + s, n)])


def _pre_kernel(tab_hbm, pk_hbm, m_hbm, z_hbm, out_hbm,
                pk0, pk1, mv0, mv1, r0, r1, acc, g0, g1, s0, s1):
    c = lax.axis_index("c")
    w = lax.axis_index("s")
    _copy_tile_rows(w, lambda s, n: z_hbm.at[pl.ds(s, n)],
                    lambda s, n: acc.at[pl.ds(s, n)])
    plsc.subcore_barrier()
    _edge_pipeline(c, w, pk_hbm, m_hbm, [tab_hbm], 16, True, acc,
                   [pk0, pk1], [mv0, mv1], [r0, r1], [g0, g1], [s0, s1],
                   PRE_SPLIT - c, c * PRE_SPLIT, NPAIRS_PRE)
    plsc.subcore_barrier()
    _copy_tile_rows(w, lambda s, n: acc.at[pl.ds(s, n)],
                    lambda s, n: out_hbm.at[pl.ds(c * N + s, n)])


@functools.partial(jax.jit, static_argnums=())
def kernel(v, l, e, e_mask, vbi, vfc, vfb, W_feat, b_feat, W_lab, b_lab,
           W_msg, b_msg, W_upd, b_upd):
    src = e[0]
    dst = e[1]
    W1 = W_msg[0:D]
    W2 = W_msg[D:2 * D]
    W3 = W_msg[2 * D:]
    Wu1 = W_upd[0:D]
    Wu2 = W_upd[D:]
    pe = jnp.asarray(_PE)
    vfc3 = vfc.reshape(NB, 1, BN)
    vbi3 = vbi.reshape(NB, 1, BN)

    pool8 = pl.pallas_call(
        _pool_body,
        grid=(NB,),
        in_specs=[
            pl.BlockSpec((1, 1, BN), lambda i: (i, 0, 0)),
            pl.BlockSpec((1, 1, BN), lambda i: (i, 0, 0)),
        ],
        out_specs=pl.BlockSpec((8, B), lambda i: (0, 0)),
        out_shape=jax.ShapeDtypeStruct((8, B), jnp.int32),
    )(vfc3, vbi3)

    x, pos16, ylo, yhi = pl.pallas_call(
        _enc_body,
        grid=(NB,),
        in_specs=[
            pl.BlockSpec((BN, 9), lambda i: (i, 0)),
            pl.BlockSpec((BN, 8), lambda i: (i, 0)),
            pl.BlockSpec((1, 1, BN), lambda i: (i, 0, 0)),
            pl.BlockSpec((1, 1, BN), lambda i: (i, 0, 0)),
            pl.BlockSpec((8, B), lambda i: (0, 0)),
            pl.BlockSpec((20, 32), lambda i: (0, 0)),
            pl.BlockSpec((6, 32), lambda i: (0, 0)),
            pl.BlockSpec((1, 32), lambda i: (0, 0)),
            pl.BlockSpec((8, 32), lambda i: (0, 0)),
            pl.BlockSpec((1, 32), lambda i: (0, 0)),
            pl.BlockSpec((D, D), lambda i: (0, 0)),
        ],
        out_specs=[
            pl.BlockSpec((BN, D), lambda i: (i, 0)),
            pl.BlockSpec((BN, 16), lambda i: (i, 0)),
            pl.BlockSpec((BN, 32), lambda i: (i, 0)),
            pl.BlockSpec((BN, 32), lambda i: (i, 0)),
        ],
        out_shape=[
            jax.ShapeDtypeStruct((N, D), jnp.float32),
            jax.ShapeDtypeStruct((N, 16), jnp.float32),
            jax.ShapeDtypeStruct((N, 32), jnp.float32),
            jax.ShapeDtypeStruct((N, 32), jnp.float32),
        ],
    )(v, l, vfc3, vbi3, pool8, pe, W_feat, b_feat.reshape(1, 32),
      W_lab, b_lab.reshape(1, 32), W2)

    z16 = jnp.zeros((N, 16), jnp.float32)
    z32 = jnp.zeros((N, 32), jnp.float32)
    pk = jnp.concatenate([
        src.reshape(NWIN, KSUB, WE),
        dst.reshape(NWIN, KSUB, WE),
    ], axis=1)
    mw = e_mask.reshape(NWIN, WWIN)

    pre = functools.partial(
        pl.kernel,
        mesh=_sc_mesh(),
        compiler_params=pltpu.CompilerParams(use_tc_tiling_on_sc=False),
        out_type=jax.ShapeDtypeStruct((2 * N, 16), jnp.float32),
        scratch_types=[
            pltpu.VMEM((2 * KSUB, WE), jnp.int32),
            pltpu.VMEM((2 * KSUB, WE), jnp.int32),
            pltpu.VMEM((WWIN,), jnp.float32),
            pltpu.VMEM((WWIN,), jnp.float32),
            pltpu.VMEM((WWIN, 16), jnp.float32),
            pltpu.VMEM((WWIN, 16), jnp.float32),
            pltpu.VMEM_SHARED((N, 16), jnp.float32),
            pltpu.SemaphoreType.DMA,
            pltpu.SemaphoreType.DMA,
            pltpu.SemaphoreType.DMA,
            pltpu.SemaphoreType.DMA,
        ],
    )(_pre_kernel)
    R2 = pre(pos16, pk, mw, z16)

    ed2, ptn = pl.pallas_call(
        _prep_body,
        grid=(NB,),
        in_specs=[
            pl.BlockSpec((BN, 16), lambda i: (i, 0)),
            pl.BlockSpec((BN, 16), lambda i: (NB + i, 0)),
            pl.BlockSpec((BN, 9), lambda i: (i, 0)),
            pl.BlockSpec((3, D), lambda i: (0, 0)),
            pl.BlockSpec((1, D), lambda i: (0, 0)),
        ],
        out_specs=[
            pl.BlockSpec((BN, 16), lambda i: (i, 0)),
            pl.BlockSpec((BN, D), lambda i: (i, 0)),
        ],
        out_shape=[
            jax.ShapeDtypeStruct((N, 16), jnp.float32),
            jax.ShapeDtypeStruct((N, D), jnp.float32),
        ],
    )(R2, R2, v, W3, b_msg.reshape(1, D))

    spmm = functools.partial(
        pl.kernel,
        mesh=_sc_mesh(),
        compiler_params=pltpu.CompilerParams(use_tc_tiling_on_sc=False),
        out_type=jax.ShapeDtypeStruct((2 * N, 32), jnp.float32),
        scratch_types=[
            pltpu.VMEM((2 * KSUB, WE), jnp.int32),
            pltpu.VMEM((2 * KSUB, WE), jnp.int32),
            pltpu.VMEM((WWIN,), jnp.float32),
            pltpu.VMEM((WWIN,), jnp.float32),
            pltpu.VMEM((WWIN, 32), jnp.float32),
            pltpu.VMEM((WWIN, 32), jnp.float32),
            pltpu.VMEM_SHARED((N, 32), jnp.float32),
            pltpu.SemaphoreType.DMA,
            pltpu.SemaphoreType.DMA,
            pltpu.SemaphoreType.DMA,
            pltpu.SemaphoreType.DMA,
        ],
    )(_spmm_kernel)

    post = pl.pallas_call(
        _post_body,
        grid=(NB,),
        in_specs=[
            pl.BlockSpec((BN, D), lambda i: (i, 0)),
            pl.BlockSpec((BN, 32), lambda i: (i, 0)),
            pl.BlockSpec((BN, 32), lambda i: (NB + i, 0)),
            pl.BlockSpec((BN, 16), lambda i: (i, 0)),
            pl.BlockSpec((BN, D), lambda i: (i, 0)),
            pl.BlockSpec((D, D), lambda i: (0, 0)),
            pl.BlockSpec((D, D), lambda i: (0, 0)),
            pl.BlockSpec((D, D), lambda i: (0, 0)),
            pl.BlockSpec((1, D), lambda i: (0, 0)),
            pl.BlockSpec((D, D), lambda i: (0, 0)),
        ],
        out_specs=[
            pl.BlockSpec((BN, D), lambda i: (i, 0)),
            pl.BlockSpec((BN, 32), lambda i: (i, 0)),
            pl.BlockSpec((BN, 32), lambda i: (i, 0)),
        ],
        out_shape=[
            jax.ShapeDtypeStruct((N, D), jnp.float32),
            jax.ShapeDtypeStruct((N, 32), jnp.float32),
            jax.ShapeDtypeStruct((N, 32), jnp.float32),
        ],
    )

    for _ in range(3):
        S2 = spmm(ylo, yhi, pk, mw, z32)
        x, ylo, yhi = post(x, S2, S2, ed2, ptn, W1, Wu1, Wu2,
                           b_upd.reshape(1, D), W2)
    return x


# speedup vs baseline: 10.2710x; 1.0432x over previous
import functools

import jax
import jax.numpy as jnp
import numpy as np
from jax import lax
from jax.experimental import pallas as pl
from jax.experimental.pallas import tpu as pltpu
from jax.experimental.pallas import tpu_sc as plsc

N = 50000
E = 800000
H = 32
D = 64
B = 16

BN = 1000
NB = N // BN
WE = 128
KSUB = 2
WWIN = KSUB * WE
NT = 16
CH = 3128
CH_LAST = N - (NT - 1) * CH


def _pe_table(d_model=32, max_len=20):
    pe = np.zeros((max_len, d_model), dtype=np.float32)
    position = np.arange(0, max_len, dtype=np.float32)[:, None]
    div_term = np.exp(np.arange(0, d_model, 2, dtype=np.float32) * (-np.log(10000.0) / d_model))
    pe[:, 0::2] = np.sin(position * div_term)
    pe[:, 1::2] = np.cos(position * div_term)
    return pe


_PE = _pe_table()


def _pool_body(vfc_ref, vbi_ref, v_ref, out_ref, pos16_ref):
    @pl.when(pl.program_id(0) == 0)
    def _():
        out_ref[...] = jnp.full((8, B), 127, jnp.int32)

    vfc_b = vfc_ref[0, 0, :]
    vbi_b = vbi_ref[0, 0, :]
    oh = vbi_b[:, None] == lax.broadcasted_iota(jnp.int32, (BN, B), 1)
    masked = jnp.where(oh, vfc_b[:, None], 127)
    colmin = jnp.min(masked, axis=0)
    out_ref[...] = jnp.minimum(out_ref[...], colmin[None, :])
    pos = v_ref[...][:, 3:6]
    ones = jnp.ones((BN, 2), jnp.float32)
    pos16_ref[...] = jnp.concatenate(
        [pos, ones, jnp.zeros((BN, 11), jnp.float32)], axis=1)


def _enc_body(v_ref, l_ref, vfc_ref, vbi_ref, pool_ref, pe_ref,
              wf_ref, bf_ref, wl_ref, bl_ref, w2_ref,
              x_ref, ylo_ref, yhi_ref):
    v_blk = v_ref[...]
    nonpos = jnp.concatenate([v_blk[:, 0:3], v_blk[:, 6:9]], axis=1)
    h = jnp.dot(nonpos, wf_ref[...], preferred_element_type=jnp.float32) + bf_ref[...]
    vfc_b = vfc_ref[0, 0, :]
    vbi_b = vbi_ref[0, 0, :]
    oh16 = vbi_b[:, None] == lax.broadcasted_iota(jnp.int32, (BN, B), 1)
    poolg = jnp.sum(jnp.where(oh16, pool_ref[0:1, :], 0), axis=1)
    lvl = vfc_b - poolg
    oh20 = (lvl[:, None] == lax.broadcasted_iota(jnp.int32, (BN, 20), 1)).astype(jnp.float32)
    pe_add = jnp.dot(oh20, pe_ref[...], preferred_element_type=jnp.float32)
    le = jnp.dot(l_ref[...], wl_ref[...], preferred_element_type=jnp.float32) + bl_ref[...]
    x = jnp.concatenate([h + pe_add, le], axis=1)
    x_ref[...] = x
    y = jnp.dot(x, w2_ref[...], preferred_element_type=jnp.float32)
    ylo_ref[...] = y[:, :32]
    yhi_ref[...] = y[:, 32:]


def _prep_body(r0_ref, r1_ref, v_ref, w3_ref, bm_ref, ed2_ref, ptn_ref):
    Rr = r0_ref[...] + r1_ref[...]
    Apos = Rr[:, 0:3]
    em = Rr[:, 3:4]
    degc = Rr[:, 4:5]
    invdeg = 1.0 / jnp.maximum(degc, 1.0)
    emn = em * invdeg
    pos = v_ref[...][:, 3:6]
    ptn = (jnp.dot(pos * em - Apos, w3_ref[...], preferred_element_type=jnp.float32)
           + em * bm_ref[...]) * invdeg
    ptn_ref[...] = ptn
    ed2_ref[...] = jnp.concatenate([emn, invdeg, jnp.zeros((BN, 14), jnp.float32)], axis=1)


def _post_body(x_ref, s0_ref, s1_ref, ed2_ref, ptn_ref,
               w1_ref, wu1_ref, wu2_ref, bu_ref, w2_ref,
               xn_ref, ylo_ref, yhi_ref):
    x = x_ref[...]
    S = jnp.concatenate([s0_ref[...], s1_ref[...]], axis=1)
    emn = ed2_ref[...][:, 0:1]
    invdeg = ed2_ref[...][:, 1:2]
    aggr = (jnp.dot(x, w1_ref[...], preferred_element_type=jnp.float32) * emn
            + S * invdeg + ptn_ref[...])
    upd = (jnp.dot(x, wu1_ref[...], preferred_element_type=jnp.float32)
           + jnp.dot(aggr, wu2_ref[...], preferred_element_type=jnp.float32)
           + bu_ref[...])
    xn = x + upd
    xn_ref[...] = xn
    y = jnp.dot(xn, w2_ref[...], preferred_element_type=jnp.float32)
    ylo_ref[...] = y[:, :32]
    yhi_ref[...] = y[:, 32:]


def _sc_mesh():
    return plsc.VectorSubcoreMesh(core_axis_name="c", subcore_axis_name="s")


def _copy_tile_rows(w, src_at, dst_at):
    @pl.when(w < NT - 1)
    def _():
        start = pl.multiple_of(w * CH, 8)
        pltpu.sync_copy(src_at(start, CH), dst_at(start, CH))

    @pl.when(w == NT - 1)
    def _():
        start = (NT - 1) * CH
        pltpu.sync_copy(src_at(start, CH_LAST), dst_at(start, CH_LAST))


def _scale_rows(rows_v, m_v, ncols, masked16=False):
    def grp(g, carry):
        m16 = m_v[pl.ds(g * 16, 16)]
        for e in range(16):
            eix = g * 16 + e
            sv = lax.broadcast_in_dim(m16[e], (16,), ())
            if masked16:
                keep = lax.iota(jnp.int32, 16) < 4
                sv = jnp.where(keep, sv, 1.0)
                rows_v[eix, pl.ds(0, 16)] = rows_v[eix, pl.ds(0, 16)] * sv
            else:
                for f0 in range(0, ncols, 16):
                    rows_v[eix, pl.ds(f0, 16)] = rows_v[eix, pl.ds(f0, 16)] * sv
        return carry
    lax.fori_loop(0, WWIN // 16, grp, 0)


def _edge_pipeline(c, w, pk_hbm, m_hbm, tables, ncols, masked16, acc,
                   pks, mvs, rows, gsems, ssems, nwin, win_off, npairs):
    def on_table(b, fn):
        for ci, tab in enumerate(tables):
            if len(tables) == 1:
                fn(tab, b)
            else:
                @pl.when(c == ci)
                def _():
                    fn(tab, b)

    def gather_rows(tab, b):
        for s in range(KSUB):
            pltpu.make_async_copy(tab.at[pks[b].at[s]],
                                  rows[b].at[pl.ds(s * WE, WE)],
                                  gsems[b]).start()

    def wait_rows(tab, b):
        for s in range(KSUB):
            pltpu.make_async_copy(tab.at[pks[b].at[s]],
                                  rows[b].at[pl.ds(s * WE, WE)],
                                  gsems[b]).wait()

    def wait_scatter(b):
        for s in range(KSUB):
            pltpu.make_async_copy(rows[b].at[pl.ds(s * WE, WE)],
                                  acc.at[pks[b].at[KSUB + s]], ssems[b]).wait()

    def stage(b, jjt):
        @pl.when(w + NT * jjt < nwin)
        def _():
            @pl.when(jjt >= 2)
            def _():
                wait_scatter(b)
            widx = win_off + w + NT * jjt
            pltpu.sync_copy(pk_hbm.at[widx], pks[b])
            pltpu.sync_copy(m_hbm.at[widx], mvs[b])
            on_table(b, gather_rows)

    def consume(b, jj):
        @pl.when(w + NT * jj < nwin)
        def _():
            on_table(b, wait_rows)
            _scale_rows(rows[b], mvs[b], ncols, masked16=masked16)
            for s in range(KSUB):
                pltpu.async_copy(rows[b].at[pl.ds(s * WE, WE)],
                                 acc.at[pks[b].at[KSUB + s]], ssems[b],
                                 add=True)

    stage(0, jnp.int32(0))

    def pair(j2, carry):
        for b in (0, 1):
            jj = 2 * j2 + b
            stage(1 - b, jj + 1)
            consume(b, jj)
        return carry

    lax.fori_loop(0, npairs, pair, 0)

    for jj in range(max(0, 2 * npairs - 4), 2 * npairs):
        @pl.when((w + NT * jj < nwin) & (w + NT * (jj + 2) >= nwin))
        def _():
            wait_scatter(jj % 2)


NWIN = E // WWIN
NPAIRS_SPMM = ((NWIN + NT - 1) // NT + 1) // 2
PRE_SPLIT = (NWIN + 1) // 2
NPAIRS_PRE = ((PRE_SPLIT + NT - 1) // NT + 1) // 2


def _spmm_kernel(ylo_hbm, yhi_hbm, pk_hbm, m_hbm, z_hbm, out_hbm,
                 pk0, pk1, mv0, mv1, r0, r1, acc, g0, g1, s0, s1):
    c = lax.axis_index("c")
    w = lax.axis_index("s")
    _copy_tile_rows(w, lambda s, n: z_hbm.at[pl.ds(s, n)],
                    lambda s, n: acc.at[pl.ds(s, n)])
    plsc.subcore_barrier()
    _edge_pipeline(c, w, pk_hbm, m_hbm, [ylo_hbm, yhi_hbm], 32, False, acc,
                   [pk0, pk1], [mv0, mv1], [r0, r1], [g0, g1], [s0, s1],
                   NWIN, 0, NPAIRS_SPMM)
    plsc.subcore_barrier()
    _copy_tile_rows(w, lambda s, n: acc.at[pl.ds(s, n)],
                    lambda s, n: out_hbm.at[pl.ds(c * N + s, n)])


def _pre_kernel(tab_hbm, pk_hbm, m_hbm, z_hbm, out_hbm,
                pk0, pk1, mv0, mv1, r0, r1, acc, g0, g1, s0, s1):
    c = lax.axis_index("c")
    w = lax.axis_index("s")
    _copy_tile_rows(w, lambda s, n: z_hbm.at[pl.ds(s, n)],
                    lambda s, n: acc.at[pl.ds(s, n)])
    plsc.subcore_barrier()
    _edge_pipeline(c, w, pk_hbm, m_hbm, [tab_hbm], 16, True, acc,
                   [pk0, pk1], [mv0, mv1], [r0, r1], [g0, g1], [s0, s1],
                   PRE_SPLIT - c, c * PRE_SPLIT, NPAIRS_PRE)
    plsc.subcore_barrier()
    _copy_tile_rows(w, lambda s, n: acc.at[pl.ds(s, n)],
                    lambda s, n: out_hbm.at[pl.ds(c * N + s, n)])


@functools.partial(jax.jit, static_argnums=())
def kernel(v, l, e, e_mask, vbi, vfc, vfb, W_feat, b_feat, W_lab, b_lab,
           W_msg, b_msg, W_upd, b_upd):
    src = e[0]
    dst = e[1]
    W1 = W_msg[0:D]
    W2 = W_msg[D:2 * D]
    W3 = W_msg[2 * D:]
    Wu1 = W_upd[0:D]
    Wu2 = W_upd[D:]
    pe = jnp.asarray(_PE)
    vfc3 = vfc.reshape(NB, 1, BN)
    vbi3 = vbi.reshape(NB, 1, BN)

    pool8, pos16 = pl.pallas_call(
        _pool_body,
        grid=(NB,),
        in_specs=[
            pl.BlockSpec((1, 1, BN), lambda i: (i, 0, 0)),
            pl.BlockSpec((1, 1, BN), lambda i: (i, 0, 0)),
            pl.BlockSpec((BN, 9), lambda i: (i, 0)),
        ],
        out_specs=[
            pl.BlockSpec((8, B), lambda i: (0, 0)),
            pl.BlockSpec((BN, 16), lambda i: (i, 0)),
        ],
        out_shape=[
            jax.ShapeDtypeStruct((8, B), jnp.int32),
            jax.ShapeDtypeStruct((N, 16), jnp.float32),
        ],
    )(vfc3, vbi3, v)

    x, ylo, yhi = pl.pallas_call(
        _enc_body,
        grid=(NB,),
        in_specs=[
            pl.BlockSpec((BN, 9), lambda i: (i, 0)),
            pl.BlockSpec((BN, 8), lambda i: (i, 0)),
            pl.BlockSpec((1, 1, BN), lambda i: (i, 0, 0)),
            pl.BlockSpec((1, 1, BN), lambda i: (i, 0, 0)),
            pl.BlockSpec((8, B), lambda i: (0, 0)),
            pl.BlockSpec((20, 32), lambda i: (0, 0)),
            pl.BlockSpec((6, 32), lambda i: (0, 0)),
            pl.BlockSpec((1, 32), lambda i: (0, 0)),
            pl.BlockSpec((8, 32), lambda i: (0, 0)),
            pl.BlockSpec((1, 32), lambda i: (0, 0)),
            pl.BlockSpec((D, D), lambda i: (0, 0)),
        ],
        out_specs=[
            pl.BlockSpec((BN, D), lambda i: (i, 0)),
            pl.BlockSpec((BN, 32), lambda i: (i, 0)),
            pl.BlockSpec((BN, 32), lambda i: (i, 0)),
        ],
        out_shape=[
            jax.ShapeDtypeStruct((N, D), jnp.float32),
            jax.ShapeDtypeStruct((N, 32), jnp.float32),
            jax.ShapeDtypeStruct((N, 32), jnp.float32),
        ],
    )(v, l, vfc3, vbi3, pool8, pe, W_feat, b_feat.reshape(1, 32),
      W_lab, b_lab.reshape(1, 32), W2)

    z16 = jnp.zeros((N, 16), jnp.float32)
    z32 = jnp.zeros((N, 32), jnp.float32)
    pk = jnp.concatenate([
        src.reshape(NWIN, KSUB, WE),
        dst.reshape(NWIN, KSUB, WE),
    ], axis=1)
    mw = e_mask.reshape(NWIN, WWIN)

    pre = functools.partial(
        pl.kernel,
        mesh=_sc_mesh(),
        compiler_params=pltpu.CompilerParams(use_tc_tiling_on_sc=False),
        out_type=jax.ShapeDtypeStruct((2 * N, 16), jnp.float32),
        scratch_types=[
            pltpu.VMEM((2 * KSUB, WE), jnp.int32),
            pltpu.VMEM((2 * KSUB, WE), jnp.int32),
            pltpu.VMEM((WWIN,), jnp.float32),
            pltpu.VMEM((WWIN,), jnp.float32),
            pltpu.VMEM((WWIN, 16), jnp.float32),
            pltpu.VMEM((WWIN, 16), jnp.float32),
            pltpu.VMEM_SHARED((N, 16), jnp.float32),
            pltpu.SemaphoreType.DMA,
            pltpu.SemaphoreType.DMA,
            pltpu.SemaphoreType.DMA,
            pltpu.SemaphoreType.DMA,
        ],
    )(_pre_kernel)
    R2 = pre(pos16, pk, mw, z16)

    ed2, ptn = pl.pallas_call(
        _prep_body,
        grid=(NB,),
        in_specs=[
            pl.BlockSpec((BN, 16), lambda i: (i, 0)),
            pl.BlockSpec((BN, 16), lambda i: (NB + i, 0)),
            pl.BlockSpec((BN, 9), lambda i: (i, 0)),
            pl.BlockSpec((3, D), lambda i: (0, 0)),
            pl.BlockSpec((1, D), lambda i: (0, 0)),
        ],
        out_specs=[
            pl.BlockSpec((BN, 16), lambda i: (i, 0)),
            pl.BlockSpec((BN, D), lambda i: (i, 0)),
        ],
        out_shape=[
            jax.ShapeDtypeStruct((N, 16), jnp.float32),
            jax.ShapeDtypeStruct((N, D), jnp.float32),
        ],
    )(R2, R2, v, W3, b_msg.reshape(1, D))

    spmm = functools.partial(
        pl.kernel,
        mesh=_sc_mesh(),
        compiler_params=pltpu.CompilerParams(use_tc_tiling_on_sc=False),
        out_type=jax.ShapeDtypeStruct((2 * N, 32), jnp.float32),
        scratch_types=[
            pltpu.VMEM((2 * KSUB, WE), jnp.int32),
            pltpu.VMEM((2 * KSUB, WE), jnp.int32),
            pltpu.VMEM((WWIN,), jnp.float32),
            pltpu.VMEM((WWIN,), jnp.float32),
            pltpu.VMEM((WWIN, 32), jnp.float32),
            pltpu.VMEM((WWIN, 32), jnp.float32),
            pltpu.VMEM_SHARED((N, 32), jnp.float32),
            pltpu.SemaphoreType.DMA,
            pltpu.SemaphoreType.DMA,
            pltpu.SemaphoreType.DMA,
            pltpu.SemaphoreType.DMA,
        ],
    )(_spmm_kernel)

    post = pl.pallas_call(
        _post_body,
        grid=(NB,),
        in_specs=[
            pl.BlockSpec((BN, D), lambda i: (i, 0)),
            pl.BlockSpec((BN, 32), lambda i: (i, 0)),
            pl.BlockSpec((BN, 32), lambda i: (NB + i, 0)),
            pl.BlockSpec((BN, 16), lambda i: (i, 0)),
            pl.BlockSpec((BN, D), lambda i: (i, 0)),
            pl.BlockSpec((D, D), lambda i: (0, 0)),
            pl.BlockSpec((D, D), lambda i: (0, 0)),
            pl.BlockSpec((D, D), lambda i: (0, 0)),
            pl.BlockSpec((1, D), lambda i: (0, 0)),
            pl.BlockSpec((D, D), lambda i: (0, 0)),
        ],
        out_specs=[
            pl.BlockSpec((BN, D), lambda i: (i, 0)),
            pl.BlockSpec((BN, 32), lambda i: (i, 0)),
            pl.BlockSpec((BN, 32), lambda i: (i, 0)),
        ],
        out_shape=[
            jax.ShapeDtypeStruct((N, D), jnp.float32),
            jax.ShapeDtypeStruct((N, 32), jnp.float32),
            jax.ShapeDtypeStruct((N, 32), jnp.float32),
        ],
    )

    for _ in range(3):
        S2 = spmm(ylo, yhi, pk, mw, z32)
        x, ylo, yhi = post(x, S2, S2, ed2, ptn, W1, Wu1, Wu2,
                           b_upd.reshape(1, D), W2)
    return x


# speedup vs baseline: 12.0451x; 1.1727x over previous
import functools

import jax
import jax.numpy as jnp
import numpy as np
from jax import lax
from jax.experimental import pallas as pl
from jax.experimental.pallas import tpu as pltpu
from jax.experimental.pallas import tpu_sc as plsc

N = 50000
E = 800000
H = 32
D = 64
B = 16

BN = 1000
NB = N // BN
WE = 128
KSUB = 2
WWIN = KSUB * WE
NT = 16
CH = 3128
CH_LAST = N - (NT - 1) * CH


def _pe_table(d_model=32, max_len=20):
    pe = np.zeros((max_len, d_model), dtype=np.float32)
    position = np.arange(0, max_len, dtype=np.float32)[:, None]
    div_term = np.exp(np.arange(0, d_model, 2, dtype=np.float32) * (-np.log(10000.0) / d_model))
    pe[:, 0::2] = np.sin(position * div_term)
    pe[:, 1::2] = np.cos(position * div_term)
    return pe


_PE = _pe_table()


def _pool_body(vfc_ref, vbi_ref, v_ref, out_ref, pos16_ref):
    @pl.when(pl.program_id(0) == 0)
    def _():
        out_ref[...] = jnp.full((8, B), 127, jnp.int32)

    vfc_b = vfc_ref[0, 0, :]
    vbi_b = vbi_ref[0, 0, :]
    oh = vbi_b[:, None] == lax.broadcasted_iota(jnp.int32, (BN, B), 1)
    masked = jnp.where(oh, vfc_b[:, None], 127)
    colmin = jnp.min(masked, axis=0)
    out_ref[...] = jnp.minimum(out_ref[...], colmin[None, :])
    pos = v_ref[...][:, 3:6]
    ones = jnp.ones((BN, 2), jnp.float32)
    pos16_ref[...] = jnp.concatenate(
        [pos, ones, jnp.zeros((BN, 11), jnp.float32)], axis=1)


def _enc_body(v_ref, l_ref, vfc_ref, vbi_ref, pool_ref, pe_ref,
              wf_ref, bf_ref, wl_ref, bl_ref, w2_ref,
              x_ref, ylo_ref, yhi_ref):
    v_blk = v_ref[...]
    nonpos = jnp.concatenate([v_blk[:, 0:3], v_blk[:, 6:9]], axis=1)
    h = jnp.dot(nonpos, wf_ref[...], preferred_element_type=jnp.float32) + bf_ref[...]
    vfc_b = vfc_ref[0, 0, :]
    vbi_b = vbi_ref[0, 0, :]
    oh16 = vbi_b[:, None] == lax.broadcasted_iota(jnp.int32, (BN, B), 1)
    poolg = jnp.sum(jnp.where(oh16, pool_ref[0:1, :], 0), axis=1)
    lvl = vfc_b - poolg
    oh20 = (lvl[:, None] == lax.broadcasted_iota(jnp.int32, (BN, 20), 1)).astype(jnp.float32)
    pe_add = jnp.dot(oh20, pe_ref[...], preferred_element_type=jnp.float32)
    le = jnp.dot(l_ref[...], wl_ref[...], preferred_element_type=jnp.float32) + bl_ref[...]
    x = jnp.concatenate([h + pe_add, le], axis=1)
    x_ref[...] = x
    y = jnp.dot(x, w2_ref[...], preferred_element_type=jnp.float32)
    ylo_ref[...] = y[:, :32]
    yhi_ref[...] = y[:, 32:]


def _prep_body(r0_ref, r1_ref, v_ref, w3_ref, bm_ref, ed2_ref, ptn_ref):
    Rr = r0_ref[...] + r1_ref[...]
    Apos = Rr[:, 0:3]
    em = Rr[:, 3:4]
    degc = Rr[:, 4:5]
    invdeg = 1.0 / jnp.maximum(degc, 1.0)
    emn = em * invdeg
    pos = v_ref[...][:, 3:6]
    ptn = (jnp.dot(pos * em - Apos, w3_ref[...], preferred_element_type=jnp.float32)
           + em * bm_ref[...]) * invdeg
    ptn_ref[...] = ptn
    ed2_ref[...] = jnp.concatenate([emn, invdeg, jnp.zeros((BN, 14), jnp.float32)], axis=1)


def _post_body(x_ref, s0_ref, s1_ref, ed2_ref, ptn_ref,
               w1_ref, wu1_ref, wu2_ref, bu_ref, w2_ref,
               xn_ref, ylo_ref, yhi_ref):
    x = x_ref[...]
    S = jnp.concatenate([s0_ref[...], s1_ref[...]], axis=1)
    emn = ed2_ref[...][:, 0:1]
    invdeg = ed2_ref[...][:, 1:2]
    aggr = (jnp.dot(x, w1_ref[...], preferred_element_type=jnp.float32) * emn
            + S * invdeg + ptn_ref[...])
    upd = (jnp.dot(x, wu1_ref[...], preferred_element_type=jnp.float32)
           + jnp.dot(aggr, wu2_ref[...], preferred_element_type=jnp.float32)
           + bu_ref[...])
    xn = x + upd
    xn_ref[...] = xn
    y = jnp.dot(xn, w2_ref[...], preferred_element_type=jnp.float32)
    ylo_ref[...] = y[:, :32]
    yhi_ref[...] = y[:, 32:]


def _sc_mesh():
    return plsc.VectorSubcoreMesh(core_axis_name="c", subcore_axis_name="s")


def _copy_tile_rows(w, src_at, dst_at):
    @pl.when(w < NT - 1)
    def _():
        start = pl.multiple_of(w * CH, 8)
        pltpu.sync_copy(src_at(start, CH), dst_at(start, CH))

    @pl.when(w == NT - 1)
    def _():
        start = (NT - 1) * CH
        pltpu.sync_copy(src_at(start, CH_LAST), dst_at(start, CH_LAST))


def _scale_rows(rows_v, pkf_v, ncols, masked16=False):
    def grp(g, carry):
        m16 = pkf_v[2 * KSUB + g // 8, pl.ds((g % 8) * 16, 16)]
        for e in range(16):
            eix = g * 16 + e
            sv = lax.broadcast_in_dim(m16[e], (16,), ())
            if masked16:
                keep = lax.iota(jnp.int32, 16) < 4
                sv = jnp.where(keep, sv, 1.0)
                rows_v[eix, pl.ds(0, 16)] = rows_v[eix, pl.ds(0, 16)] * sv
            else:
                for f0 in range(0, ncols, 16):
                    rows_v[eix, pl.ds(f0, 16)] = rows_v[eix, pl.ds(f0, 16)] * sv
        return carry
    lax.fori_loop(0, WWIN // 16, grp, 0)


def _edge_pipeline(c, w, pk_hbm, tables, ncols, masked16, acc,
                   pkfs, pkis, rows, gsems, ssems, nwin, win_off, npairs):
    def on_table(b, fn):
        for ci, tab in enumerate(tables):
            if len(tables) == 1:
                fn(tab, b)
            else:
                @pl.when(c == ci)
                def _():
                    fn(tab, b)

    def gather_rows(tab, b):
        for s in range(KSUB):
            pltpu.make_async_copy(tab.at[pkis[b].at[s]],
                                  rows[b].at[pl.ds(s * WE, WE)],
                                  gsems[b]).start()

    def wait_rows(tab, b):
        for s in range(KSUB):
            pltpu.make_async_copy(tab.at[pkis[b].at[s]],
                                  rows[b].at[pl.ds(s * WE, WE)],
                                  gsems[b]).wait()

    def wait_scatter(b):
        for s in range(KSUB):
            pltpu.make_async_copy(rows[b].at[pl.ds(s * WE, WE)],
                                  acc.at[pkis[b].at[KSUB + s]], ssems[b]).wait()

    def stage(b, jjt):
        @pl.when(w + NT * jjt < nwin)
        def _():
            @pl.when(jjt >= 2)
            def _():
                wait_scatter(b)
            widx = win_off + w + NT * jjt
            pltpu.sync_copy(pk_hbm.at[widx], pkfs[b])
            for r in range(2 * KSUB):
                for g in range(WE // 16):
                    pkis[b][r, pl.ds(g * 16, 16)] = (
                        pkfs[b][r, pl.ds(g * 16, 16)].astype(jnp.int32))
            on_table(b, gather_rows)

    def consume(b, jj):
        @pl.when(w + NT * jj < nwin)
        def _():
            on_table(b, wait_rows)
            _scale_rows(rows[b], pkfs[b], ncols, masked16=masked16)
            for s in range(KSUB):
                pltpu.async_copy(rows[b].at[pl.ds(s * WE, WE)],
                                 acc.at[pkis[b].at[KSUB + s]], ssems[b],
                                 add=True)

    stage(0, jnp.int32(0))

    def pair(j2, carry):
        for b in (0, 1):
            jj = 2 * j2 + b
            stage(1 - b, jj + 1)
            consume(b, jj)
        return carry

    lax.fori_loop(0, npairs, pair, 0)

    for jj in range(max(0, 2 * npairs - 4), 2 * npairs):
        @pl.when((w + NT * jj < nwin) & (w + NT * (jj + 2) >= nwin))
        def _():
            wait_scatter(jj % 2)


NWIN = E // WWIN
NPAIRS_SPMM = ((NWIN + NT - 1) // NT + 1) // 2
PRE_SPLIT = (NWIN + 1) // 2
NPAIRS_PRE = ((PRE_SPLIT + NT - 1) // NT + 1) // 2


def _spmm_kernel(ylo_hbm, yhi_hbm, pk_hbm, z_hbm, out_hbm,
                 pkf0, pkf1, pki0, pki1, r0, r1, acc, g0, g1, s0, s1):
    c = lax.axis_index("c")
    w = lax.axis_index("s")
    _copy_tile_rows(w, lambda s, n: z_hbm.at[pl.ds(s, n)],
                    lambda s, n: acc.at[pl.ds(s, n)])
    plsc.subcore_barrier()
    _edge_pipeline(c, w, pk_hbm, [ylo_hbm, yhi_hbm], 32, False, acc,
                   [pkf0, pkf1], [pki0, pki1], [r0, r1], [g0, g1], [s0, s1],
                   NWIN, 0, NPAIRS_SPMM)
    plsc.subcore_barrier()
    _copy_tile_rows(w, lambda s, n: acc.at[pl.ds(s, n)],
                    lambda s, n: out_hbm.at[pl.ds(c * N + s, n)])


def _pre_kernel(tab_hbm, pk_hbm, z_hbm, out_hbm,
                pkf0, pkf1, pki0, pki1, r0, r1, acc, g0, g1, s0, s1):
    c = lax.axis_index("c")
    w = lax.axis_index("s")
    _copy_tile_rows(w, lambda s, n: z_hbm.at[pl.ds(s, n)],
                    lambda s, n: acc.at[pl.ds(s, n)])
    plsc.subcore_barrier()
    _edge_pipeline(c, w, pk_hbm, [tab_hbm], 16, True, acc,
                   [pkf0, pkf1], [pki0, pki1], [r0, r1], [g0, g1], [s0, s1],
                   PRE_SPLIT - c, c * PRE_SPLIT, NPAIRS_PRE)
    plsc.subcore_barrier()
    _copy_tile_rows(w, lambda s, n: acc.at[pl.ds(s, n)],
                    lambda s, n: out_hbm.at[pl.ds(c * N + s, n)])


@functools.partial(jax.jit, static_argnums=())
def kernel(v, l, e, e_mask, vbi, vfc, vfb, W_feat, b_feat, W_lab, b_lab,
           W_msg, b_msg, W_upd, b_upd):
    src = e[0]
    dst = e[1]
    W1 = W_msg[0:D]
    W2 = W_msg[D:2 * D]
    W3 = W_msg[2 * D:]
    Wu1 = W_upd[0:D]
    Wu2 = W_upd[D:]
    pe = jnp.asarray(_PE)
    vfc3 = vfc.reshape(NB, 1, BN)
    vbi3 = vbi.reshape(NB, 1, BN)

    pool8, pos16 = pl.pallas_call(
        _pool_body,
        grid=(NB,),
        in_specs=[
            pl.BlockSpec((1, 1, BN), lambda i: (i, 0, 0)),
            pl.BlockSpec((1, 1, BN), lambda i: (i, 0, 0)),
            pl.BlockSpec((BN, 9), lambda i: (i, 0)),
        ],
        out_specs=[
            pl.BlockSpec((8, B), lambda i: (0, 0)),
            pl.BlockSpec((BN, 16), lambda i: (i, 0)),
        ],
        out_shape=[
            jax.ShapeDtypeStruct((8, B), jnp.int32),
            jax.ShapeDtypeStruct((N, 16), jnp.float32),
        ],
    )(vfc3, vbi3, v)

    x, ylo, yhi = pl.pallas_call(
        _enc_body,
        grid=(NB,),
        in_specs=[
            pl.BlockSpec((BN, 9), lambda i: (i, 0)),
            pl.BlockSpec((BN, 8), lambda i: (i, 0)),
            pl.BlockSpec((1, 1, BN), lambda i: (i, 0, 0)),
            pl.BlockSpec((1, 1, BN), lambda i: (i, 0, 0)),
            pl.BlockSpec((8, B), lambda i: (0, 0)),
            pl.BlockSpec((20, 32), lambda i: (0, 0)),
            pl.BlockSpec((6, 32), lambda i: (0, 0)),
            pl.BlockSpec((1, 32), lambda i: (0, 0)),
            pl.BlockSpec((8, 32), lambda i: (0, 0)),
            pl.BlockSpec((1, 32), lambda i: (0, 0)),
            pl.BlockSpec((D, D), lambda i: (0, 0)),
        ],
        out_specs=[
            pl.BlockSpec((BN, D), lambda i: (i, 0)),
            pl.BlockSpec((BN, 32), lambda i: (i, 0)),
            pl.BlockSpec((BN, 32), lambda i: (i, 0)),
        ],
        out_shape=[
            jax.ShapeDtypeStruct((N, D), jnp.float32),
            jax.ShapeDtypeStruct((N, 32), jnp.float32),
            jax.ShapeDtypeStruct((N, 32), jnp.float32),
        ],
    )(v, l, vfc3, vbi3, pool8, pe, W_feat, b_feat.reshape(1, 32),
      W_lab, b_lab.reshape(1, 32), W2)

    z16 = jnp.zeros((N, 16), jnp.float32)
    z32 = jnp.zeros((N, 32), jnp.float32)
    pk = jnp.concatenate([
        src.astype(jnp.float32).reshape(NWIN, KSUB, WE),
        dst.astype(jnp.float32).reshape(NWIN, KSUB, WE),
        e_mask.reshape(NWIN, KSUB, WE),
    ], axis=1)

    pre = functools.partial(
        pl.kernel,
        mesh=_sc_mesh(),
        compiler_params=pltpu.CompilerParams(use_tc_tiling_on_sc=False),
        out_type=jax.ShapeDtypeStruct((2 * N, 16), jnp.float32),
        scratch_types=[
            pltpu.VMEM((3 * KSUB, WE), jnp.float32),
            pltpu.VMEM((3 * KSUB, WE), jnp.float32),
            pltpu.VMEM((2 * KSUB, WE), jnp.int32),
            pltpu.VMEM((2 * KSUB, WE), jnp.int32),
            pltpu.VMEM((WWIN, 16), jnp.float32),
            pltpu.VMEM((WWIN, 16), jnp.float32),
            pltpu.VMEM_SHARED((N, 16), jnp.float32),
            pltpu.SemaphoreType.DMA,
            pltpu.SemaphoreType.DMA,
            pltpu.SemaphoreType.DMA,
            pltpu.SemaphoreType.DMA,
        ],
    )(_pre_kernel)
    R2 = pre(pos16, pk, z16)

    ed2, ptn = pl.pallas_call(
        _prep_body,
        grid=(NB,),
        in_specs=[
            pl.BlockSpec((BN, 16), lambda i: (i, 0)),
            pl.BlockSpec((BN, 16), lambda i: (NB + i, 0)),
            pl.BlockSpec((BN, 9), lambda i: (i, 0)),
            pl.BlockSpec((3, D), lambda i: (0, 0)),
            pl.BlockSpec((1, D), lambda i: (0, 0)),
        ],
        out_specs=[
            pl.BlockSpec((BN, 16), lambda i: (i, 0)),
            pl.BlockSpec((BN, D), lambda i: (i, 0)),
        ],
        out_shape=[
            jax.ShapeDtypeStruct((N, 16), jnp.float32),
            jax.ShapeDtypeStruct((N, D), jnp.float32),
        ],
    )(R2, R2, v, W3, b_msg.reshape(1, D))

    spmm = functools.partial(
        pl.kernel,
        mesh=_sc_mesh(),
        compiler_params=pltpu.CompilerParams(use_tc_tiling_on_sc=False),
        out_type=jax.ShapeDtypeStruct((2 * N, 32), jnp.float32),
        scratch_types=[
            pltpu.VMEM((3 * KSUB, WE), jnp.float32),
            pltpu.VMEM((3 * KSUB, WE), jnp.float32),
            pltpu.VMEM((2 * KSUB, WE), jnp.int32),
            pltpu.VMEM((2 * KSUB, WE), jnp.int32),
            pltpu.VMEM((WWIN, 32), jnp.float32),
            pltpu.VMEM((WWIN, 32), jnp.float32),
            pltpu.VMEM_SHARED((N, 32), jnp.float32),
            pltpu.SemaphoreType.DMA,
            pltpu.SemaphoreType.DMA,
            pltpu.SemaphoreType.DMA,
            pltpu.SemaphoreType.DMA,
        ],
    )(_spmm_kernel)

    post = pl.pallas_call(
        _post_body,
        grid=(NB,),
        in_specs=[
            pl.BlockSpec((BN, D), lambda i: (i, 0)),
            pl.BlockSpec((BN, 32), lambda i: (i, 0)),
            pl.BlockSpec((BN, 32), lambda i: (NB + i, 0)),
            pl.BlockSpec((BN, 16), lambda i: (i, 0)),
            pl.BlockSpec((BN, D), lambda i: (i, 0)),
            pl.BlockSpec((D, D), lambda i: (0, 0)),
            pl.BlockSpec((D, D), lambda i: (0, 0)),
            pl.BlockSpec((D, D), lambda i: (0, 0)),
            pl.BlockSpec((1, D), lambda i: (0, 0)),
            pl.BlockSpec((D, D), lambda i: (0, 0)),
        ],
        out_specs=[
            pl.BlockSpec((BN, D), lambda i: (i, 0)),
            pl.BlockSpec((BN, 32), lambda i: (i, 0)),
            pl.BlockSpec((BN, 32), lambda i: (i, 0)),
        ],
        out_shape=[
            jax.ShapeDtypeStruct((N, D), jnp.float32),
            jax.ShapeDtypeStruct((N, 32), jnp.float32),
            jax.ShapeDtypeStruct((N, 32), jnp.float32),
        ],
    )

    for _ in range(3):
        S2 = spmm(ylo, yhi, pk, z32)
        x, ylo, yhi = post(x, S2, S2, ed2, ptn, W1, Wu1, Wu2,
                           b_upd.reshape(1, D), W2)
    return x
